# Initial kernel scaffold; baseline (speedup 1.0000x reference)
import jax
import jax.numpy as jnp
from jax.experimental import pallas as pl


def kernel(q_pts, s_pts, s_feats, neighb_inds, Wq, bq, Wk, bk, Wv, bv, Wd1, bd1, g_d1, be_d1, Wd2, bd2, g_a0, be_a0, Wa1, ba1, g_a1, be_a1, Wa2, ba2):
    raise NotImplementedError("write your pallas kernel here")


# speedup vs baseline: 1.9215x; 1.9215x over previous
import functools

import jax
import jax.numpy as jnp
from jax import lax
from jax.experimental import pallas as pl
from jax.experimental.pallas import tpu as pltpu
from jax.experimental.pallas import tpu_sc as plsc

N = 10000
H = 32
C = 128
CPG = 16
EDGES = N * H
BM = 200
BE = BM * H
GRID = N // BM
EPS = 1e-5


_NW = 32
_EPW = EDGES // _NW
_CH = 80
_NCH = _EPW // _CH
_QROWS = 400
_QCH = _QROWS // _CH


def _sc_gather(inds_flat, inds0, ktab, vtab, qtab, px, py, pz):
  mesh = plsc.VectorSubcoreMesh(core_axis_name="c", subcore_axis_name="s")

  @functools.partial(
      pl.kernel,
      out_type=[
          jax.ShapeDtypeStruct((EDGES, C), jnp.float32),
          jax.ShapeDtypeStruct((EDGES, C), jnp.float32),
          jax.ShapeDtypeStruct((EDGES,), jnp.float32),
          jax.ShapeDtypeStruct((EDGES,), jnp.float32),
          jax.ShapeDtypeStruct((EDGES,), jnp.float32),
          jax.ShapeDtypeStruct((N, C), jnp.float32),
      ],
      mesh=mesh,
      compiler_params=pltpu.CompilerParams(needs_layout_passes=False),
      scratch_types=[
          pltpu.VMEM((_CH,), jnp.int32),
          pltpu.VMEM((_CH, C), jnp.float32),
          pltpu.VMEM((_CH, C), jnp.float32),
          pltpu.VMEM((N,), jnp.float32),
          pltpu.VMEM((N,), jnp.float32),
          pltpu.VMEM((N,), jnp.float32),
          pltpu.VMEM((_CH,), jnp.float32),
          pltpu.VMEM((_CH,), jnp.float32),
          pltpu.VMEM((_CH,), jnp.float32),
          pltpu.SemaphoreType.DMA,
          pltpu.SemaphoreType.DMA,
      ],
  )
  def gather_kernel(inds_hbm, inds0_hbm, ktab_hbm, vtab_hbm, qtab_hbm,
                    px_hbm, py_hbm, pz_hbm,
                    nk_out, nv_out, npx_out, npy_out, npz_out, qsel_out,
                    idx_v, kbuf, vbuf, xtab, ytab, ztab, xbuf, ybuf, zbuf,
                    semk, semv):
    wid = lax.axis_index("s") * 2 + lax.axis_index("c")
    base = wid * _EPW

    pltpu.sync_copy(px_hbm, xtab)
    pltpu.sync_copy(py_hbm, ytab)
    pltpu.sync_copy(pz_hbm, ztab)

    def body(i, _):
      off = base + i * _CH
      pltpu.sync_copy(inds_hbm.at[pl.ds(off, _CH)], idx_v)
      ck = pltpu.async_copy(ktab_hbm.at[idx_v], kbuf, semk)
      cv = pltpu.async_copy(vtab_hbm.at[idx_v], vbuf, semv)
      for j in range(_CH // 16):
        sl = pl.ds(j * 16, 16)
        idx16 = idx_v[sl]
        xbuf[sl] = plsc.load_gather(xtab, [idx16])
        ybuf[sl] = plsc.load_gather(ytab, [idx16])
        zbuf[sl] = plsc.load_gather(ztab, [idx16])
      pltpu.sync_copy(xbuf, npx_out.at[pl.ds(off, _CH)])
      pltpu.sync_copy(ybuf, npy_out.at[pl.ds(off, _CH)])
      pltpu.sync_copy(zbuf, npz_out.at[pl.ds(off, _CH)])
      ck.wait()
      cv.wait()
      pltpu.sync_copy(kbuf, nk_out.at[pl.ds(off, _CH)])
      pltpu.sync_copy(vbuf, nv_out.at[pl.ds(off, _CH)])
      return 0

    lax.fori_loop(0, _NCH, body, 0)

    @pl.when(wid < _NW - 7)
    def _():
      qbase = wid * _QROWS

      def qbody(i, _):
        off = qbase + i * _CH
        pltpu.sync_copy(inds0_hbm.at[pl.ds(off, _CH)], idx_v)
        pltpu.async_copy(qtab_hbm.at[idx_v], kbuf, semk).wait()
        pltpu.sync_copy(kbuf, qsel_out.at[pl.ds(off, _CH)])
        return 0

      lax.fori_loop(0, _QCH, qbody, 0)

  return gather_kernel(inds_flat, inds0, ktab, vtab, qtab, px, py, pz)


def _qkv_body(s_ref, w_ref, b_ref, o_ref):
  o_ref[...] = (
      jnp.dot(s_ref[...], w_ref[...], preferred_element_type=jnp.float32)
      + b_ref[...]
  )


def _qkv(s_feats, w_all, b_all):
  bm = 2000
  return pl.pallas_call(
      _qkv_body,
      grid=(N // bm,),
      in_specs=[
          pl.BlockSpec((bm, C), lambda i: (i, 0)),
          pl.BlockSpec((C, 3 * C), lambda i: (0, 0)),
          pl.BlockSpec((1, 3 * C), lambda i: (0, 0)),
      ],
      out_specs=pl.BlockSpec((bm, 3 * C), lambda i: (i, 0)),
      out_shape=jax.ShapeDtypeStruct((N, 3 * C), jnp.float32),
  )(s_feats, w_all, b_all)


def _leaky(x):
  return jnp.where(x >= 0, x, 0.1 * x)


def _y1_raw(npx_ref, npy_ref, npz_ref, qx_ref, qy_ref, qz_ref, w1x_ref,
            w1y_ref, w1z_ref):
  nx = npx_ref[...] - qx_ref[...]
  ny = npy_ref[...] - qy_ref[...]
  nz = npz_ref[...] - qz_ref[...]
  y1 = (
      nx[:, :, None] * w1x_ref[...][None, :, :]
      + ny[:, :, None] * w1y_ref[...][None, :, :]
      + nz[:, :, None] * w1z_ref[...][None, :, :]
  )
  return y1.reshape(BE, C // 4)


def _geom(y1, sc1_ref, sh1_ref, wd2_ref, bd2_ref):
  hg = _leaky(y1 * sc1_ref[...] + sh1_ref[...])
  return (
      jnp.dot(hg, wd2_ref[...], preferred_element_type=jnp.float32)
      + bd2_ref[...]
  )


_GEO_SPECS = [
    pl.BlockSpec((BM, H), lambda i: (i, 0)),
    pl.BlockSpec((BM, H), lambda i: (i, 0)),
    pl.BlockSpec((BM, H), lambda i: (i, 0)),
    pl.BlockSpec((BM, 1), lambda i: (i, 0)),
    pl.BlockSpec((BM, 1), lambda i: (i, 0)),
    pl.BlockSpec((BM, 1), lambda i: (i, 0)),
    pl.BlockSpec((1, C // 4), lambda i: (0, 0)),
    pl.BlockSpec((1, C // 4), lambda i: (0, 0)),
    pl.BlockSpec((1, C // 4), lambda i: (0, 0)),
]


def _passA_body(npx_ref, npy_ref, npz_ref, qx_ref, qy_ref, qz_ref, w1x_ref,
                w1y_ref, w1z_ref, bd1_ref, sum_ref):
  y1 = _y1_raw(npx_ref, npy_ref, npz_ref, qx_ref, qy_ref, qz_ref, w1x_ref,
               w1y_ref, w1z_ref) + bd1_ref[...]
  s1 = jnp.sum(y1, axis=0)
  s2 = jnp.sum(y1 * y1, axis=0)

  @pl.when(pl.program_id(0) == 0)
  def _():
    sum_ref[...] = jnp.zeros_like(sum_ref)

  sum_ref[0, :] += s1
  sum_ref[1, :] += s2


def _passA(geo_args, bd1):
  return pl.pallas_call(
      _passA_body,
      grid=(GRID,),
      in_specs=_GEO_SPECS + [pl.BlockSpec((1, C // 4), lambda i: (0, 0))],
      out_specs=pl.BlockSpec((2, C // 4), lambda i: (0, 0)),
      out_shape=jax.ShapeDtypeStruct((2, C // 4), jnp.float32),
  )(*geo_args, bd1)


def _passB_body(nk_ref, npx_ref, npy_ref, npz_ref, qx_ref, qy_ref, qz_ref,
                w1x_ref, w1y_ref, w1z_ref, qsel_ref, sc1_ref, sh1_ref,
                wd2_ref, bd2_ref, sum_ref):
  y1 = _y1_raw(npx_ref, npy_ref, npz_ref, qx_ref, qy_ref, qz_ref, w1x_ref,
               w1y_ref, w1z_ref)
  geom = _geom(y1, sc1_ref, sh1_ref, wd2_ref, bd2_ref)
  qk = (qsel_ref[...][:, None, :] - nk_ref[...]).reshape(BE, C) - geom
  s1 = jnp.sum(qk, axis=0)
  s2 = jnp.sum(qk * qk, axis=0)

  @pl.when(pl.program_id(0) == 0)
  def _():
    sum_ref[...] = jnp.zeros_like(sum_ref)

  sum_ref[0, :] += s1
  sum_ref[1, :] += s2


def _passB(nk3, geo_args, q_sel, sc1, sh1, wd2, bd2):
  return pl.pallas_call(
      _passB_body,
      grid=(GRID,),
      in_specs=[pl.BlockSpec((BM, H, C), lambda i: (i, 0, 0))] + _GEO_SPECS
      + [
          pl.BlockSpec((BM, C), lambda i: (i, 0)),
          pl.BlockSpec((1, C // 4), lambda i: (0, 0)),
          pl.BlockSpec((1, C // 4), lambda i: (0, 0)),
          pl.BlockSpec((C // 4, C), lambda i: (0, 0)),
          pl.BlockSpec((1, C), lambda i: (0, 0)),
      ],
      out_specs=pl.BlockSpec((2, C), lambda i: (0, 0)),
      out_shape=jax.ShapeDtypeStruct((2, C), jnp.float32),
  )(nk3, *geo_args, q_sel, sc1, sh1, wd2, bd2)


def _passC_body(nk_ref, npx_ref, npy_ref, npz_ref, qx_ref, qy_ref, qz_ref,
                w1x_ref, w1y_ref, w1z_ref, qsel_ref, sc1_ref, sh1_ref,
                wd2_ref, bd2_ref, sc2_ref, sh2_ref, wa1_ref, ba1_ref,
                y3_ref, sum_ref):
  y1 = _y1_raw(npx_ref, npy_ref, npz_ref, qx_ref, qy_ref, qz_ref, w1x_ref,
               w1y_ref, w1z_ref)
  geom = _geom(y1, sc1_ref, sh1_ref, wd2_ref, bd2_ref)
  qk = (qsel_ref[...][:, None, :] - nk_ref[...]).reshape(BE, C) - geom
  a = _leaky(qk * sc2_ref[...] + sh2_ref[...])
  y3 = (
      jnp.dot(a, wa1_ref[...], preferred_element_type=jnp.float32)
      + ba1_ref[...]
  )
  y3_ref[...] = y3.reshape(BM, H, CPG)
  s1 = jnp.sum(y3, axis=0)
  s2 = jnp.sum(y3 * y3, axis=0)

  @pl.when(pl.program_id(0) == 0)
  def _():
    sum_ref[...] = jnp.zeros_like(sum_ref)

  sum_ref[0, :] += s1
  sum_ref[1, :] += s2


def _passC(nk3, geo_args, q_sel, sc1, sh1, wd2, bd2, sc2, sh2, wa1, ba1):
  return pl.pallas_call(
      _passC_body,
      grid=(GRID,),
      in_specs=[pl.BlockSpec((BM, H, C), lambda i: (i, 0, 0))] + _GEO_SPECS
      + [
          pl.BlockSpec((BM, C), lambda i: (i, 0)),
          pl.BlockSpec((1, C // 4), lambda i: (0, 0)),
          pl.BlockSpec((1, C // 4), lambda i: (0, 0)),
          pl.BlockSpec((C // 4, C), lambda i: (0, 0)),
          pl.BlockSpec((1, C), lambda i: (0, 0)),
          pl.BlockSpec((1, C), lambda i: (0, 0)),
          pl.BlockSpec((1, C), lambda i: (0, 0)),
          pl.BlockSpec((C, CPG), lambda i: (0, 0)),
          pl.BlockSpec((1, CPG), lambda i: (0, 0)),
      ],
      out_specs=[
          pl.BlockSpec((BM, H, CPG), lambda i: (i, 0, 0)),
          pl.BlockSpec((2, CPG), lambda i: (0, 0)),
      ],
      out_shape=[
          jax.ShapeDtypeStruct((N, H, CPG), jnp.float32),
          jax.ShapeDtypeStruct((2, CPG), jnp.float32),
      ],
  )(nk3, *geo_args, q_sel, sc1, sh1, wd2, bd2, sc2, sh2, wa1, ba1)


def _passD_body(nv_ref, npx_ref, npy_ref, npz_ref, qx_ref, qy_ref, qz_ref,
                w1x_ref, w1y_ref, w1z_ref, y3_ref, sc1_ref, sh1_ref,
                wd2_ref, bd2_ref, sc3_ref, sh3_ref, wa2_ref, ba2_ref,
                out_ref):
  y1 = _y1_raw(npx_ref, npy_ref, npz_ref, qx_ref, qy_ref, qz_ref, w1x_ref,
               w1y_ref, w1z_ref)
  geom = _geom(y1, sc1_ref, sh1_ref, wd2_ref, bd2_ref)
  vmg = nv_ref[...] - geom.reshape(BM, H, C)
  a2 = _leaky(y3_ref[...].reshape(BE, CPG) * sc3_ref[...] + sh3_ref[...])
  a3 = (
      jnp.dot(a2, wa2_ref[...], preferred_element_type=jnp.float32)
      + ba2_ref[...]
  ).reshape(BM, H, CPG)
  m = jnp.max(a3, axis=1, keepdims=True)
  e = jnp.exp(a3 - m)
  s = jnp.sum(e, axis=1, keepdims=True)
  attn = e / s
  attn_t = jnp.concatenate([attn] * (C // CPG), axis=2)
  out_ref[...] = jnp.sum(vmg * attn_t, axis=1)


def _passD(nv3, geo_args, y3, sc1, sh1, wd2, bd2, sc3, sh3, wa2, ba2):
  return pl.pallas_call(
      _passD_body,
      grid=(GRID,),
      in_specs=[pl.BlockSpec((BM, H, C), lambda i: (i, 0, 0))] + _GEO_SPECS
      + [
          pl.BlockSpec((BM, H, CPG), lambda i: (i, 0, 0)),
          pl.BlockSpec((1, C // 4), lambda i: (0, 0)),
          pl.BlockSpec((1, C // 4), lambda i: (0, 0)),
          pl.BlockSpec((C // 4, C), lambda i: (0, 0)),
          pl.BlockSpec((1, C), lambda i: (0, 0)),
          pl.BlockSpec((1, CPG), lambda i: (0, 0)),
          pl.BlockSpec((1, CPG), lambda i: (0, 0)),
          pl.BlockSpec((CPG, CPG), lambda i: (0, 0)),
          pl.BlockSpec((1, CPG), lambda i: (0, 0)),
      ],
      out_specs=pl.BlockSpec((BM, C), lambda i: (i, 0)),
      out_shape=jax.ShapeDtypeStruct((N, C), jnp.float32),
  )(nv3, *geo_args, y3, sc1, sh1, wd2, bd2, sc3, sh3, wa2, ba2)


def _bn_affine(sums, gamma, beta, bias):
  mean = sums[0] / EDGES
  var = sums[1] / EDGES - mean * mean
  rstd = lax.rsqrt(var + EPS)
  sc = rstd * gamma
  sh = (bias - mean) * sc + beta
  return sc.reshape(1, -1), sh.reshape(1, -1)


def kernel(q_pts, s_pts, s_feats, neighb_inds, Wq, bq, Wk, bk, Wv, bv, Wd1,
           bd1, g_d1, be_d1, Wd2, bd2, g_a0, be_a0, Wa1, ba1, g_a1, be_a1,
           Wa2, ba2):
  inds_flat = neighb_inds.reshape(-1)
  inds0 = neighb_inds[:, 0]
  px, py, pz = s_pts[:, 0], s_pts[:, 1], s_pts[:, 2]
  qx, qy, qz = q_pts[:, 0:1], q_pts[:, 1:2], q_pts[:, 2:3]
  w1x = Wd1[0:1, :]
  w1y = Wd1[1:2, :]
  w1z = Wd1[2:3, :]
  w_all = jnp.concatenate([Wq, Wk, Wv], axis=1)
  b_all = jnp.concatenate([bq, bk, bv]).reshape(1, 3 * C)

  qkv = _qkv(s_feats, w_all, b_all)
  qtab = qkv[:, :C]
  ktab = qkv[:, C:2 * C]
  vtab = qkv[:, 2 * C:]

  nk, nv, npx, npy, npz, q_sel = _sc_gather(
      inds_flat, inds0, ktab, vtab, qtab, px, py, pz)
  nk3 = nk.reshape(N, H, C)
  nv3 = nv.reshape(N, H, C)
  geo_args = (npx.reshape(N, H), npy.reshape(N, H), npz.reshape(N, H),
              qx, qy, qz, w1x, w1y, w1z)

  sumsA = _passA(geo_args, bd1.reshape(1, -1))
  sc1, sh1 = _bn_affine(sumsA, g_d1, be_d1, bd1)
  bd2r = bd2.reshape(1, C)

  sumsB = _passB(nk3, geo_args, q_sel, sc1, sh1, Wd2, bd2r)
  mean2 = sumsB[0] / EDGES
  var2 = sumsB[1] / EDGES - mean2 * mean2
  rstd2 = lax.rsqrt(var2 + EPS)
  sc2 = (rstd2 * g_a0).reshape(1, C)
  sh2 = (be_a0 - mean2 * rstd2 * g_a0).reshape(1, C)

  y3, sumsC = _passC(nk3, geo_args, q_sel, sc1, sh1, Wd2, bd2r, sc2, sh2,
                     Wa1, ba1.reshape(1, CPG))
  sc3, sh3 = _bn_affine(sumsC, g_a1, be_a1, jnp.zeros((CPG,), jnp.float32))

  out = _passD(nv3, geo_args, y3, sc1, sh1, Wd2, bd2r, sc3, sh3, Wa2,
               ba2.reshape(1, CPG))
  return out


# speedup vs baseline: 2.5940x; 1.3500x over previous
import functools

import jax
import jax.numpy as jnp
from jax import lax
from jax.experimental import pallas as pl
from jax.experimental.pallas import tpu as pltpu
from jax.experimental.pallas import tpu_sc as plsc

N = 10000
H = 32
C = 128
CPG = 16
EDGES = N * H
BM = 200
BE = BM * H
GRID = N // BM
EPS = 1e-5


_NW = 32
_EPW = EDGES // _NW
_CH = 80
_NCH = _EPW // _CH
_QROWS = 400
_QCH = _QROWS // _CH


def _sc_gather(inds_flat, inds0, ktab, vtab, qtab, px, py, pz, qx, qy, qz):
  mesh = plsc.VectorSubcoreMesh(core_axis_name="c", subcore_axis_name="s")

  @functools.partial(
      pl.kernel,
      out_type=[
          jax.ShapeDtypeStruct((EDGES, C), jnp.float32),
          jax.ShapeDtypeStruct((EDGES, C), jnp.float32),
          jax.ShapeDtypeStruct((EDGES * 4,), jnp.float32),
          jax.ShapeDtypeStruct((N, C), jnp.float32),
      ],
      mesh=mesh,
      compiler_params=pltpu.CompilerParams(needs_layout_passes=False),
      scratch_types=[
          pltpu.VMEM((_CH,), jnp.int32),
          pltpu.VMEM((_CH, C), jnp.float32),
          pltpu.VMEM((_CH, C), jnp.float32),
          pltpu.VMEM((N,), jnp.float32),
          pltpu.VMEM((N,), jnp.float32),
          pltpu.VMEM((N,), jnp.float32),
          pltpu.VMEM((N,), jnp.float32),
          pltpu.VMEM((N,), jnp.float32),
          pltpu.VMEM((N,), jnp.float32),
          pltpu.VMEM((_CH * 4,), jnp.float32),
          pltpu.SemaphoreType.DMA,
          pltpu.SemaphoreType.DMA,
      ],
  )
  def gather_kernel(inds_hbm, inds0_hbm, ktab_hbm, vtab_hbm, qtab_hbm,
                    px_hbm, py_hbm, pz_hbm, qx_hbm, qy_hbm, qz_hbm,
                    nk_out, nv_out, nbr_out, qsel_out,
                    idx_v, kbuf, vbuf, xtab, ytab, ztab, qxtab, qytab,
                    qztab, pbuf, semk, semv):
    wid = lax.axis_index("s") * 2 + lax.axis_index("c")
    base = wid * _EPW
    iota16 = lax.iota(jnp.int32, 16)
    zero16 = jnp.zeros((16,), jnp.float32)

    pltpu.sync_copy(px_hbm, xtab)
    pltpu.sync_copy(py_hbm, ytab)
    pltpu.sync_copy(pz_hbm, ztab)
    pltpu.sync_copy(qx_hbm, qxtab)
    pltpu.sync_copy(qy_hbm, qytab)
    pltpu.sync_copy(qz_hbm, qztab)

    def body(i, _):
      off = base + i * _CH
      pltpu.sync_copy(inds_hbm.at[pl.ds(off, _CH)], idx_v)
      ck = pltpu.async_copy(ktab_hbm.at[idx_v], kbuf, semk)
      cv = pltpu.async_copy(vtab_hbm.at[idx_v], vbuf, semv)
      for j in range(_CH // 16):
        sl = pl.ds(j * 16, 16)
        idx16 = idx_v[sl]
        rowi = lax.shift_right_logical(
            jnp.full((16,), off + j * 16, jnp.int32) + iota16, 5)
        lid = iota16 * 4 + (j * 64)
        plsc.store_scatter(
            pbuf, [lid],
            plsc.load_gather(xtab, [idx16])
            - plsc.load_gather(qxtab, [rowi]))
        plsc.store_scatter(
            pbuf, [lid + 1],
            plsc.load_gather(ytab, [idx16])
            - plsc.load_gather(qytab, [rowi]))
        plsc.store_scatter(
            pbuf, [lid + 2],
            plsc.load_gather(ztab, [idx16])
            - plsc.load_gather(qztab, [rowi]))
        plsc.store_scatter(pbuf, [lid + 3], zero16)
      pltpu.sync_copy(pbuf, nbr_out.at[pl.ds(off * 4, _CH * 4)])
      ck.wait()
      cv.wait()
      pltpu.sync_copy(kbuf, nk_out.at[pl.ds(off, _CH)])
      pltpu.sync_copy(vbuf, nv_out.at[pl.ds(off, _CH)])
      return 0

    lax.fori_loop(0, _NCH, body, 0)

    @pl.when(wid < _NW - 7)
    def _():
      qbase = wid * _QROWS

      def qbody(i, _):
        off = qbase + i * _CH
        pltpu.sync_copy(inds0_hbm.at[pl.ds(off, _CH)], idx_v)
        pltpu.async_copy(qtab_hbm.at[idx_v], kbuf, semk).wait()
        pltpu.sync_copy(kbuf, qsel_out.at[pl.ds(off, _CH)])
        return 0

      lax.fori_loop(0, _QCH, qbody, 0)

  return gather_kernel(inds_flat, inds0, ktab, vtab, qtab, px, py, pz,
                       qx, qy, qz)


def _qkv_body(s_ref, w_ref, b_ref, o_ref):
  o_ref[...] = (
      jnp.dot(s_ref[...], w_ref[...], preferred_element_type=jnp.float32)
      + b_ref[...]
  )


def _qkv(s_feats, w_all, b_all):
  bm = 2000
  return pl.pallas_call(
      _qkv_body,
      grid=(N // bm,),
      in_specs=[
          pl.BlockSpec((bm, C), lambda i: (i, 0)),
          pl.BlockSpec((C, 3 * C), lambda i: (0, 0)),
          pl.BlockSpec((1, 3 * C), lambda i: (0, 0)),
      ],
      out_specs=pl.BlockSpec((bm, 3 * C), lambda i: (i, 0)),
      out_shape=jax.ShapeDtypeStruct((N, 3 * C), jnp.float32),
  )(s_feats, w_all, b_all)


def _leaky(x):
  return jnp.where(x >= 0, x, 0.1 * x)


def _geom(nbr_ref, sc1_ref, sh1_ref, wd1_ref, wd2_ref, bd2_ref):
  y1 = jnp.dot(nbr_ref[...], wd1_ref[...],
               preferred_element_type=jnp.float32)
  hg = _leaky(y1 * sc1_ref[...] + sh1_ref[...])
  return (
      jnp.dot(hg, wd2_ref[...], preferred_element_type=jnp.float32)
      + bd2_ref[...]
  )


def _qk_edges(nk_ref, qsel_ref, geom):
  qsel_e = jnp.broadcast_to(qsel_ref[...][:, None, :],
                            (BM, H, C)).reshape(BE, C)
  return qsel_e - nk_ref[...] - geom


def _passA_body(nbr_ref, wd1_ref, bd1_ref, sum_ref):
  y1 = (
      jnp.dot(nbr_ref[...], wd1_ref[...], preferred_element_type=jnp.float32)
      + bd1_ref[...]
  )
  s1 = jnp.sum(y1, axis=0)
  s2 = jnp.sum(y1 * y1, axis=0)

  @pl.when(pl.program_id(0) == 0)
  def _():
    sum_ref[...] = jnp.zeros_like(sum_ref)

  sum_ref[0, :] += s1
  sum_ref[1, :] += s2


def _passA(nbr4, wd1p, bd1):
  return pl.pallas_call(
      _passA_body,
      grid=(GRID,),
      in_specs=[
          pl.BlockSpec((BE, 4), lambda i: (i, 0)),
          pl.BlockSpec((4, C // 4), lambda i: (0, 0)),
          pl.BlockSpec((1, C // 4), lambda i: (0, 0)),
      ],
      out_specs=pl.BlockSpec((2, C // 4), lambda i: (0, 0)),
      out_shape=jax.ShapeDtypeStruct((2, C // 4), jnp.float32),
  )(nbr4, wd1p, bd1)


_SMALL = lambda shape: pl.BlockSpec(shape, lambda i: (0, 0))


def _passB_body(nk_ref, nbr_ref, qsel_ref, wd1_ref, sc1_ref, sh1_ref,
                wd2_ref, bd2_ref, sum_ref):
  geom = _geom(nbr_ref, sc1_ref, sh1_ref, wd1_ref, wd2_ref, bd2_ref)
  qk = _qk_edges(nk_ref, qsel_ref, geom)
  s1 = jnp.sum(qk, axis=0)
  s2 = jnp.sum(qk * qk, axis=0)

  @pl.when(pl.program_id(0) == 0)
  def _():
    sum_ref[...] = jnp.zeros_like(sum_ref)

  sum_ref[0, :] += s1
  sum_ref[1, :] += s2


def _passB(nk, nbr4, q_sel, wd1p, sc1, sh1, wd2, bd2):
  return pl.pallas_call(
      _passB_body,
      grid=(GRID,),
      in_specs=[
          pl.BlockSpec((BE, C), lambda i: (i, 0)),
          pl.BlockSpec((BE, 4), lambda i: (i, 0)),
          pl.BlockSpec((BM, C), lambda i: (i, 0)),
          _SMALL((4, C // 4)),
          _SMALL((1, C // 4)),
          _SMALL((1, C // 4)),
          _SMALL((C // 4, C)),
          _SMALL((1, C)),
      ],
      out_specs=pl.BlockSpec((2, C), lambda i: (0, 0)),
      out_shape=jax.ShapeDtypeStruct((2, C), jnp.float32),
  )(nk, nbr4, q_sel, wd1p, sc1, sh1, wd2, bd2)


def _passC_body(nk_ref, nbr_ref, qsel_ref, wd1_ref, sc1_ref, sh1_ref,
                wd2_ref, bd2_ref, sc2_ref, sh2_ref, wa1_ref, ba1_ref,
                y3_ref, sum_ref):
  geom = _geom(nbr_ref, sc1_ref, sh1_ref, wd1_ref, wd2_ref, bd2_ref)
  qk = _qk_edges(nk_ref, qsel_ref, geom)
  a = _leaky(qk * sc2_ref[...] + sh2_ref[...])
  y3 = (
      jnp.dot(a, wa1_ref[...], preferred_element_type=jnp.float32)
      + ba1_ref[...]
  )
  y3_ref[...] = y3
  s1 = jnp.sum(y3, axis=0)
  s2 = jnp.sum(y3 * y3, axis=0)

  @pl.when(pl.program_id(0) == 0)
  def _():
    sum_ref[...] = jnp.zeros_like(sum_ref)

  sum_ref[0, :] += s1
  sum_ref[1, :] += s2


def _passC(nk, nbr4, q_sel, wd1p, sc1, sh1, wd2, bd2, sc2, sh2, wa1, ba1):
  return pl.pallas_call(
      _passC_body,
      grid=(GRID,),
      in_specs=[
          pl.BlockSpec((BE, C), lambda i: (i, 0)),
          pl.BlockSpec((BE, 4), lambda i: (i, 0)),
          pl.BlockSpec((BM, C), lambda i: (i, 0)),
          _SMALL((4, C // 4)),
          _SMALL((1, C // 4)),
          _SMALL((1, C // 4)),
          _SMALL((C // 4, C)),
          _SMALL((1, C)),
          _SMALL((1, C)),
          _SMALL((1, C)),
          _SMALL((C, CPG)),
          _SMALL((1, CPG)),
      ],
      out_specs=[
          pl.BlockSpec((BE, CPG), lambda i: (i, 0)),
          pl.BlockSpec((2, CPG), lambda i: (0, 0)),
      ],
      out_shape=[
          jax.ShapeDtypeStruct((EDGES, CPG), jnp.float32),
          jax.ShapeDtypeStruct((2, CPG), jnp.float32),
      ],
  )(nk, nbr4, q_sel, wd1p, sc1, sh1, wd2, bd2, sc2, sh2, wa1, ba1)


def _passD_body(nv_ref, nbr_ref, y3_ref, wd1_ref, sc1_ref, sh1_ref,
                wd2_ref, bd2_ref, sc3_ref, sh3_ref, wa2_ref, ba2_ref,
                tile_ref, out_ref):
  geom = _geom(nbr_ref, sc1_ref, sh1_ref, wd1_ref, wd2_ref, bd2_ref)
  vmg = nv_ref[...] - geom
  a2 = _leaky(y3_ref[...] * sc3_ref[...] + sh3_ref[...])
  a3 = (
      jnp.dot(a2, wa2_ref[...], preferred_element_type=jnp.float32)
      + ba2_ref[...]
  ).reshape(BM, H, CPG)
  m = jnp.max(a3, axis=1, keepdims=True)
  e = jnp.exp(a3 - m)
  s = jnp.sum(e, axis=1)
  et = jnp.dot(e.reshape(BE, CPG), tile_ref[...],
               preferred_element_type=jnp.float32)
  raw = jnp.sum((vmg * et).reshape(BM, H, C), axis=1)
  rst = jnp.dot(1.0 / s, tile_ref[...],
                preferred_element_type=jnp.float32)
  out_ref[...] = raw * rst


def _passD(nv, nbr4, y3, wd1p, sc1, sh1, wd2, bd2, sc3, sh3, wa2, ba2,
           tile_mat):
  return pl.pallas_call(
      _passD_body,
      grid=(GRID,),
      in_specs=[
          pl.BlockSpec((BE, C), lambda i: (i, 0)),
          pl.BlockSpec((BE, 4), lambda i: (i, 0)),
          pl.BlockSpec((BE, CPG), lambda i: (i, 0)),
          _SMALL((4, C // 4)),
          _SMALL((1, C // 4)),
          _SMALL((1, C // 4)),
          _SMALL((C // 4, C)),
          _SMALL((1, C)),
          _SMALL((1, CPG)),
          _SMALL((1, CPG)),
          _SMALL((CPG, CPG)),
          _SMALL((1, CPG)),
          _SMALL((CPG, C)),
      ],
      out_specs=pl.BlockSpec((BM, C), lambda i: (i, 0)),
      out_shape=jax.ShapeDtypeStruct((N, C), jnp.float32),
  )(nv, nbr4, y3, wd1p, sc1, sh1, wd2, bd2, sc3, sh3, wa2, ba2, tile_mat)


def _bn_affine(sums, gamma, beta, bias):
  mean = sums[0] / EDGES
  var = sums[1] / EDGES - mean * mean
  rstd = lax.rsqrt(var + EPS)
  sc = rstd * gamma
  sh = (bias - mean) * sc + beta
  return sc.reshape(1, -1), sh.reshape(1, -1)


def kernel(q_pts, s_pts, s_feats, neighb_inds, Wq, bq, Wk, bk, Wv, bv, Wd1,
           bd1, g_d1, be_d1, Wd2, bd2, g_a0, be_a0, Wa1, ba1, g_a1, be_a1,
           Wa2, ba2):
  inds_flat = neighb_inds.reshape(-1)
  inds0 = neighb_inds[:, 0]
  px, py, pz = s_pts[:, 0], s_pts[:, 1], s_pts[:, 2]
  qx, qy, qz = q_pts[:, 0], q_pts[:, 1], q_pts[:, 2]
  wd1p = jnp.pad(Wd1, ((0, 1), (0, 0)))
  w_all = jnp.concatenate([Wq, Wk, Wv], axis=1)
  b_all = jnp.concatenate([bq, bk, bv]).reshape(1, 3 * C)
  tile_mat = jnp.tile(jnp.eye(CPG, dtype=jnp.float32), (1, C // CPG))

  qkv = _qkv(s_feats, w_all, b_all)
  qtab = qkv[:, :C]
  ktab = qkv[:, C:2 * C]
  vtab = qkv[:, 2 * C:]

  nk, nv, nbr_flat, q_sel = _sc_gather(inds_flat, inds0, ktab, vtab, qtab,
                                       px, py, pz, qx, qy, qz)
  nbr4 = nbr_flat.reshape(EDGES, 4)

  sumsA = _passA(nbr4, wd1p, bd1.reshape(1, -1))
  sc1, sh1 = _bn_affine(sumsA, g_d1, be_d1, bd1)
  bd2r = bd2.reshape(1, C)

  sumsB = _passB(nk, nbr4, q_sel, wd1p, sc1, sh1, Wd2, bd2r)
  mean2 = sumsB[0] / EDGES
  var2 = sumsB[1] / EDGES - mean2 * mean2
  rstd2 = lax.rsqrt(var2 + EPS)
  sc2 = (rstd2 * g_a0).reshape(1, C)
  sh2 = (be_a0 - mean2 * rstd2 * g_a0).reshape(1, C)

  y3, sumsC = _passC(nk, nbr4, q_sel, wd1p, sc1, sh1, Wd2, bd2r, sc2, sh2,
                     Wa1, ba1.reshape(1, CPG))
  sc3, sh3 = _bn_affine(sumsC, g_a1, be_a1, jnp.zeros((CPG,), jnp.float32))

  out = _passD(nv, nbr4, y3, wd1p, sc1, sh1, Wd2, bd2r, sc3, sh3, Wa2,
               ba2.reshape(1, CPG), tile_mat)
  return out


# speedup vs baseline: 3.2981x; 1.2714x over previous
import functools

import jax
import jax.numpy as jnp
from jax import lax
from jax.experimental import pallas as pl
from jax.experimental.pallas import tpu as pltpu
from jax.experimental.pallas import tpu_sc as plsc

N = 10000
H = 32
C = 128
CPG = 16
EDGES = N * H
BM = 200
BE = BM * H
GRID = N // BM
EPS = 1e-5


_NW = 32
_EPW = EDGES // _NW
_CH = 80
_NCH = _EPW // _CH
_QROWS = 400
_QCH = _QROWS // _CH


def _sc_gather(inds_flat, inds0, ktab, vtab, qtab, px, py, pz, qx, qy, qz):
  mesh = plsc.VectorSubcoreMesh(core_axis_name="c", subcore_axis_name="s")

  @functools.partial(
      pl.kernel,
      out_type=[
          jax.ShapeDtypeStruct((EDGES, C), jnp.float32),
          jax.ShapeDtypeStruct((EDGES, C), jnp.float32),
          jax.ShapeDtypeStruct((EDGES * 4,), jnp.float32),
          jax.ShapeDtypeStruct((N, C), jnp.float32),
      ],
      mesh=mesh,
      compiler_params=pltpu.CompilerParams(needs_layout_passes=False),
      scratch_types=[
          pltpu.VMEM((_CH,), jnp.int32),
          pltpu.VMEM((_CH, C), jnp.float32),
          pltpu.VMEM((_CH, C), jnp.float32),
          pltpu.VMEM((N,), jnp.float32),
          pltpu.VMEM((N,), jnp.float32),
          pltpu.VMEM((N,), jnp.float32),
          pltpu.VMEM((N,), jnp.float32),
          pltpu.VMEM((N,), jnp.float32),
          pltpu.VMEM((N,), jnp.float32),
          pltpu.VMEM((_CH * 4,), jnp.float32),
          pltpu.SemaphoreType.DMA,
          pltpu.SemaphoreType.DMA,
      ],
  )
  def gather_kernel(inds_hbm, inds0_hbm, ktab_hbm, vtab_hbm, qtab_hbm,
                    px_hbm, py_hbm, pz_hbm, qx_hbm, qy_hbm, qz_hbm,
                    nk_out, nv_out, nbr_out, qsel_out,
                    idx_v, kbuf, vbuf, xtab, ytab, ztab, qxtab, qytab,
                    qztab, pbuf, semk, semv):
    wid = lax.axis_index("s") * 2 + lax.axis_index("c")
    base = wid * _EPW
    iota16 = lax.iota(jnp.int32, 16)
    zero16 = jnp.zeros((16,), jnp.float32)

    pltpu.sync_copy(px_hbm, xtab)
    pltpu.sync_copy(py_hbm, ytab)
    pltpu.sync_copy(pz_hbm, ztab)
    pltpu.sync_copy(qx_hbm, qxtab)
    pltpu.sync_copy(qy_hbm, qytab)
    pltpu.sync_copy(qz_hbm, qztab)

    def body(i, _):
      off = base + i * _CH
      pltpu.sync_copy(inds_hbm.at[pl.ds(off, _CH)], idx_v)
      ck = pltpu.async_copy(ktab_hbm.at[idx_v], kbuf, semk)
      cv = pltpu.async_copy(vtab_hbm.at[idx_v], vbuf, semv)
      for j in range(_CH // 16):
        sl = pl.ds(j * 16, 16)
        idx16 = idx_v[sl]
        rowi = lax.shift_right_logical(
            jnp.full((16,), off + j * 16, jnp.int32) + iota16, 5)
        lid = iota16 * 4 + (j * 64)
        plsc.store_scatter(
            pbuf, [lid],
            plsc.load_gather(xtab, [idx16])
            - plsc.load_gather(qxtab, [rowi]))
        plsc.store_scatter(
            pbuf, [lid + 1],
            plsc.load_gather(ytab, [idx16])
            - plsc.load_gather(qytab, [rowi]))
        plsc.store_scatter(
            pbuf, [lid + 2],
            plsc.load_gather(ztab, [idx16])
            - plsc.load_gather(qztab, [rowi]))
        plsc.store_scatter(pbuf, [lid + 3], zero16)
      pltpu.sync_copy(pbuf, nbr_out.at[pl.ds(off * 4, _CH * 4)])
      ck.wait()
      cv.wait()
      pltpu.sync_copy(kbuf, nk_out.at[pl.ds(off, _CH)])
      pltpu.sync_copy(vbuf, nv_out.at[pl.ds(off, _CH)])
      return 0

    lax.fori_loop(0, _NCH, body, 0)

    @pl.when(wid < _NW - 7)
    def _():
      qbase = wid * _QROWS

      def qbody(i, _):
        off = qbase + i * _CH
        pltpu.sync_copy(inds0_hbm.at[pl.ds(off, _CH)], idx_v)
        pltpu.async_copy(qtab_hbm.at[idx_v], kbuf, semk).wait()
        pltpu.sync_copy(kbuf, qsel_out.at[pl.ds(off, _CH)])
        return 0

      lax.fori_loop(0, _QCH, qbody, 0)

  return gather_kernel(inds_flat, inds0, ktab, vtab, qtab, px, py, pz,
                       qx, qy, qz)


def _qkv_body(s_ref, w_ref, b_ref, q_ref, k_ref, v_ref):
  acc = (
      jnp.dot(s_ref[...], w_ref[...], preferred_element_type=jnp.float32)
      + b_ref[...]
  )
  q_ref[...] = acc[:, :C]
  k_ref[...] = acc[:, C:2 * C]
  v_ref[...] = acc[:, 2 * C:]


def _qkv(s_feats, w_all, b_all):
  bm = 2000
  spec = pl.BlockSpec((bm, C), lambda i: (i, 0))
  return pl.pallas_call(
      _qkv_body,
      grid=(N // bm,),
      in_specs=[
          pl.BlockSpec((bm, C), lambda i: (i, 0)),
          pl.BlockSpec((C, 3 * C), lambda i: (0, 0)),
          pl.BlockSpec((1, 3 * C), lambda i: (0, 0)),
      ],
      out_specs=[spec, spec, spec],
      out_shape=[jax.ShapeDtypeStruct((N, C), jnp.float32)] * 3,
  )(s_feats, w_all, b_all)


def _leaky(x):
  return jnp.where(x >= 0, x, 0.1 * x)


def _edge_expand(wide, mask_ref, gmat_ref):
  rows = wide.shape[0]
  grp = BE // rows
  brd = jnp.broadcast_to(wide[:, None, :], (rows, grp, C))
  masked = (brd * mask_ref[...][None, :, :]).reshape(BE, C)
  return jnp.dot(masked, gmat_ref[...], preferred_element_type=jnp.float32)


def _geom(nbr_ref, sc1_ref, sh1_ref, m32_ref, g1_ref, wd2_ref, bd2_ref):
  y1 = _edge_expand(nbr_ref[...], m32_ref, g1_ref)
  hg = _leaky(y1 * sc1_ref[...] + sh1_ref[...])
  return (
      jnp.dot(hg, wd2_ref[...], preferred_element_type=jnp.float32)
      + bd2_ref[...]
  )


def _qk_edges(nk_ref, qsel_ref, geom):
  qsel_e = jnp.broadcast_to(qsel_ref[...][:, None, :],
                            (BM, H, C)).reshape(BE, C)
  return qsel_e - nk_ref[...] - geom


def _passA_body(nbr_ref, m32_ref, g1_ref, bd1_ref, sum_ref):
  y1 = _edge_expand(nbr_ref[...], m32_ref, g1_ref) + bd1_ref[...]
  s1 = jnp.sum(y1, axis=0)
  s2 = jnp.sum(y1 * y1, axis=0)

  @pl.when(pl.program_id(0) == 0)
  def _():
    sum_ref[...] = jnp.zeros_like(sum_ref)

  sum_ref[0, :] += s1
  sum_ref[1, :] += s2


def _passA(nbr4, m32, g1, bd1):
  return pl.pallas_call(
      _passA_body,
      grid=(GRID,),
      in_specs=[
          pl.BlockSpec((BE * 4 // C, C), lambda i: (i, 0)),
          pl.BlockSpec((H, C), lambda i: (0, 0)),
          pl.BlockSpec((C, C // 4), lambda i: (0, 0)),
          pl.BlockSpec((1, C // 4), lambda i: (0, 0)),
      ],
      out_specs=pl.BlockSpec((2, C // 4), lambda i: (0, 0)),
      out_shape=jax.ShapeDtypeStruct((2, C // 4), jnp.float32),
  )(nbr4, m32, g1, bd1)


_SMALL = lambda shape: pl.BlockSpec(shape, lambda i: (0, 0))


def _passB_body(nk_ref, nbr_ref, qsel_ref, m32_ref, g1_ref, sc1_ref,
                sh1_ref, wd2_ref, bd2_ref, sum_ref):
  geom = _geom(nbr_ref, sc1_ref, sh1_ref, m32_ref, g1_ref, wd2_ref, bd2_ref)
  qk = _qk_edges(nk_ref, qsel_ref, geom)
  s1 = jnp.sum(qk, axis=0)
  s2 = jnp.sum(qk * qk, axis=0)

  @pl.when(pl.program_id(0) == 0)
  def _():
    sum_ref[...] = jnp.zeros_like(sum_ref)

  sum_ref[0, :] += s1
  sum_ref[1, :] += s2


def _passB(nk, nbr4, q_sel, m32, g1, sc1, sh1, wd2, bd2):
  return pl.pallas_call(
      _passB_body,
      grid=(GRID,),
      in_specs=[
          pl.BlockSpec((BE, C), lambda i: (i, 0)),
          pl.BlockSpec((BE * 4 // C, C), lambda i: (i, 0)),
          pl.BlockSpec((BM, C), lambda i: (i, 0)),
          _SMALL((H, C)),
          _SMALL((C, C // 4)),
          _SMALL((1, C // 4)),
          _SMALL((1, C // 4)),
          _SMALL((C // 4, C)),
          _SMALL((1, C)),
      ],
      out_specs=pl.BlockSpec((2, C), lambda i: (0, 0)),
      out_shape=jax.ShapeDtypeStruct((2, C), jnp.float32),
  )(nk, nbr4, q_sel, m32, g1, sc1, sh1, wd2, bd2)


def _passC_body(nk_ref, nbr_ref, qsel_ref, m32_ref, g1_ref, sc1_ref,
                sh1_ref, wd2_ref, bd2_ref, sc2_ref, sh2_ref, wa1_ref,
                ba1_ref, tile_ref, m8_ref, y3_ref, sum_ref):
  geom = _geom(nbr_ref, sc1_ref, sh1_ref, m32_ref, g1_ref, wd2_ref, bd2_ref)
  qk = _qk_edges(nk_ref, qsel_ref, geom)
  a = _leaky(qk * sc2_ref[...] + sh2_ref[...])
  y3 = (
      jnp.dot(a, wa1_ref[...], preferred_element_type=jnp.float32)
      + ba1_ref[...]
  )
  y3sp = (
      jnp.dot(y3, tile_ref[...], preferred_element_type=jnp.float32)
      .reshape(BM, H, C) * m8_ref[...][None, :, :]
  ).reshape(BE * CPG // C, C // CPG, C)
  y3_ref[...] = jnp.sum(y3sp, axis=1)
  s1 = jnp.sum(y3, axis=0)
  s2 = jnp.sum(y3 * y3, axis=0)

  @pl.when(pl.program_id(0) == 0)
  def _():
    sum_ref[...] = jnp.zeros_like(sum_ref)

  sum_ref[0, :] += s1
  sum_ref[1, :] += s2


def _passC(nk, nbr4, q_sel, m32, g1, sc1, sh1, wd2, bd2, sc2, sh2, wa1,
           ba1, tile_mat, m8):
  return pl.pallas_call(
      _passC_body,
      grid=(GRID,),
      in_specs=[
          pl.BlockSpec((BE, C), lambda i: (i, 0)),
          pl.BlockSpec((BE * 4 // C, C), lambda i: (i, 0)),
          pl.BlockSpec((BM, C), lambda i: (i, 0)),
          _SMALL((H, C)),
          _SMALL((C, C // 4)),
          _SMALL((1, C // 4)),
          _SMALL((1, C // 4)),
          _SMALL((C // 4, C)),
          _SMALL((1, C)),
          _SMALL((1, C)),
          _SMALL((1, C)),
          _SMALL((C, CPG)),
          _SMALL((1, CPG)),
          _SMALL((CPG, C)),
          _SMALL((H, C)),
      ],
      out_specs=[
          pl.BlockSpec((BE * CPG // C, C), lambda i: (i, 0)),
          pl.BlockSpec((2, CPG), lambda i: (0, 0)),
      ],
      out_shape=[
          jax.ShapeDtypeStruct((EDGES * CPG // C, C), jnp.float32),
          jax.ShapeDtypeStruct((2, CPG), jnp.float32),
      ],
  )(nk, nbr4, q_sel, m32, g1, sc1, sh1, wd2, bd2, sc2, sh2, wa1,
    ba1, tile_mat, m8)


def _lane_butterfly(x, op):
  for sh in (CPG, 2 * CPG, 4 * CPG):
    x = op(x, pltpu.roll(x, sh, 1))
  return x


def _passD_body(nv_ref, nbr_ref, y3_ref, m32_ref, g1_ref, sc1_ref, sh1_ref,
                wd2_ref, bd2_ref, sc3_ref, sh3_ref, w2bd_ref, ba2_ref,
                m8_ref, gbig_ref, out_ref):
  geom = _geom(nbr_ref, sc1_ref, sh1_ref, m32_ref, g1_ref, wd2_ref, bd2_ref)
  vmg = nv_ref[...] - geom
  y3w = y3_ref[...]
  a2w = _leaky(y3w * sc3_ref[...] + sh3_ref[...])
  a3w = (
      jnp.dot(a2w, w2bd_ref[...], preferred_element_type=jnp.float32)
      + ba2_ref[...]
  )
  a34 = a3w.reshape(BM, H // 8, C)
  mx = _lane_butterfly(jnp.max(a34, axis=1), jnp.maximum)
  eb = jnp.exp(a3w - jnp.broadcast_to(
      mx[:, None, :], (BM, H // 8, C)).reshape(BM * H // 8, C))
  s = _lane_butterfly(jnp.sum(eb.reshape(BM, H // 8, C), axis=1),
                      jnp.add)
  rows = BM * H // 8
  ebrd = jnp.broadcast_to(eb[:, None, :], (rows, 8, C))
  emsk = (ebrd.reshape(BM, H, C) * m8_ref[...][None, :, :]).reshape(BE, C)
  et = jnp.dot(emsk, gbig_ref[...], preferred_element_type=jnp.float32)
  raw = jnp.sum((vmg * et).reshape(BM, H, C), axis=1)
  out_ref[...] = raw * (1.0 / s)


def _passD(nv, nbr4, y3, m32, g1, sc1, sh1, wd2, bd2, sc3t, sh3t, w2bd,
           ba2t, m8, gbig):
  return pl.pallas_call(
      _passD_body,
      grid=(GRID,),
      in_specs=[
          pl.BlockSpec((BE, C), lambda i: (i, 0)),
          pl.BlockSpec((BE * 4 // C, C), lambda i: (i, 0)),
          pl.BlockSpec((BE * CPG // C, C), lambda i: (i, 0)),
          _SMALL((H, C)),
          _SMALL((C, C // 4)),
          _SMALL((1, C // 4)),
          _SMALL((1, C // 4)),
          _SMALL((C // 4, C)),
          _SMALL((1, C)),
          _SMALL((1, C)),
          _SMALL((1, C)),
          _SMALL((C, C)),
          _SMALL((1, C)),
          _SMALL((H, C)),
          _SMALL((C, C)),
      ],
      out_specs=pl.BlockSpec((BM, C), lambda i: (i, 0)),
      out_shape=jax.ShapeDtypeStruct((N, C), jnp.float32),
  )(nv, nbr4, y3, m32, g1, sc1, sh1, wd2, bd2, sc3t, sh3t, w2bd, ba2t,
    m8, gbig)


def _bn_affine(sums, gamma, beta, bias):
  mean = sums[0] / EDGES
  var = sums[1] / EDGES - mean * mean
  rstd = lax.rsqrt(var + EPS)
  sc = rstd * gamma
  sh = (bias - mean) * sc + beta
  return sc.reshape(1, -1), sh.reshape(1, -1)


def kernel(q_pts, s_pts, s_feats, neighb_inds, Wq, bq, Wk, bk, Wv, bv, Wd1,
           bd1, g_d1, be_d1, Wd2, bd2, g_a0, be_a0, Wa1, ba1, g_a1, be_a1,
           Wa2, ba2):
  inds_flat = neighb_inds.reshape(-1)
  inds0 = neighb_inds[:, 0]
  px, py, pz = s_pts[:, 0], s_pts[:, 1], s_pts[:, 2]
  qx, qy, qz = q_pts[:, 0], q_pts[:, 1], q_pts[:, 2]
  wd1p = jnp.pad(Wd1, ((0, 1), (0, 0)))
  w_all = jnp.concatenate([Wq, Wk, Wv], axis=1)
  b_all = jnp.concatenate([bq, bk, bv]).reshape(1, 3 * C)
  tile_mat = jnp.tile(jnp.eye(CPG, dtype=jnp.float32), (1, C // CPG))
  g1 = jnp.tile(wd1p, (H, 1))
  lane = jnp.arange(C)
  m32 = (lane[None, :] // 4 == jnp.arange(H)[:, None]).astype(jnp.float32)
  m8 = (lane[None, :] // CPG
        == (jnp.arange(H) % 8)[:, None]).astype(jnp.float32)
  gbig = jnp.tile(jnp.eye(CPG, dtype=jnp.float32), (C // CPG, C // CPG))
  w2bd = jnp.kron(jnp.eye(C // CPG, dtype=jnp.float32), Wa2)

  qtab, ktab, vtab = _qkv(s_feats, w_all, b_all)

  nk, nv, nbr_flat, q_sel = _sc_gather(inds_flat, inds0, ktab, vtab, qtab,
                                       px, py, pz, qx, qy, qz)
  nbr4 = nbr_flat.reshape(EDGES * 4 // C, C)

  sumsA = _passA(nbr4, m32, g1, bd1.reshape(1, -1))
  sc1, sh1 = _bn_affine(sumsA, g_d1, be_d1, bd1)
  bd2r = bd2.reshape(1, C)

  sumsB = _passB(nk, nbr4, q_sel, m32, g1, sc1, sh1, Wd2, bd2r)
  mean2 = sumsB[0] / EDGES
  var2 = sumsB[1] / EDGES - mean2 * mean2
  rstd2 = lax.rsqrt(var2 + EPS)
  sc2 = (rstd2 * g_a0).reshape(1, C)
  sh2 = (be_a0 - mean2 * rstd2 * g_a0).reshape(1, C)

  y3, sumsC = _passC(nk, nbr4, q_sel, m32, g1, sc1, sh1, Wd2, bd2r, sc2,
                     sh2, Wa1, ba1.reshape(1, CPG), tile_mat, m8)
  sc3, sh3 = _bn_affine(sumsC, g_a1, be_a1, jnp.zeros((CPG,), jnp.float32))
  sc3t = jnp.tile(sc3, (1, C // CPG))
  sh3t = jnp.tile(sh3, (1, C // CPG))
  ba2t = jnp.tile(ba2, C // CPG).reshape(1, C)

  out = _passD(nv, nbr4, y3, m32, g1, sc1, sh1, Wd2, bd2r, sc3t, sh3t,
               w2bd, ba2t, m8, gbig)
  return out


# speedup vs baseline: 3.6322x; 1.1013x over previous
import functools

import jax
import jax.numpy as jnp
from jax import lax
from jax.experimental import pallas as pl
from jax.experimental.pallas import tpu as pltpu
from jax.experimental.pallas import tpu_sc as plsc

N = 10000
H = 32
C = 128
CPG = 16
EDGES = N * H
BM = 200
BE = BM * H
GRID = N // BM
EPS = 1e-5


_NW = 32
_EPW = EDGES // _NW
_CH = 80
_NCH = _EPW // _CH
_QROWS = 400
_QCH = _QROWS // _CH


def _sc_gather(inds_flat, inds0, kvtab, qtab, px, py, pz, qx, qy, qz):
  mesh = plsc.VectorSubcoreMesh(core_axis_name="c", subcore_axis_name="s")

  @functools.partial(
      pl.kernel,
      out_type=[
          jax.ShapeDtypeStruct((EDGES, C), jnp.float32),
          jax.ShapeDtypeStruct((EDGES * 4,), jnp.float32),
          jax.ShapeDtypeStruct((N, C), jnp.float32),
      ],
      mesh=mesh,
      compiler_params=pltpu.CompilerParams(needs_layout_passes=False),
      scratch_types=[
          pltpu.VMEM((_CH,), jnp.int32),
          pltpu.VMEM((_CH, C), jnp.float32),
          pltpu.VMEM((_CH, C), jnp.float32),
          pltpu.VMEM((N,), jnp.float32),
          pltpu.VMEM((N,), jnp.float32),
          pltpu.VMEM((N,), jnp.float32),
          pltpu.VMEM((N,), jnp.float32),
          pltpu.VMEM((N,), jnp.float32),
          pltpu.VMEM((N,), jnp.float32),
          pltpu.VMEM((_CH * 4,), jnp.float32),
          pltpu.SemaphoreType.DMA,
          pltpu.SemaphoreType.DMA,
      ],
  )
  def gather_kernel(inds_hbm, inds0_hbm, kvtab_hbm, qtab_hbm,
                    px_hbm, py_hbm, pz_hbm, qx_hbm, qy_hbm, qz_hbm,
                    nkv_out, nbr_out, qsel_out,
                    idx_v, kvbuf, qbuf, xtab, ytab, ztab, qxtab,
                    qytab, qztab, pbuf, semk, semv):
    wid = lax.axis_index("s") * 2 + lax.axis_index("c")
    base = wid * _EPW
    iota16 = lax.iota(jnp.int32, 16)
    zero16 = jnp.zeros((16,), jnp.float32)

    pltpu.sync_copy(px_hbm, xtab)
    pltpu.sync_copy(py_hbm, ytab)
    pltpu.sync_copy(pz_hbm, ztab)
    pltpu.sync_copy(qx_hbm, qxtab)
    pltpu.sync_copy(qy_hbm, qytab)
    pltpu.sync_copy(qz_hbm, qztab)

    def body(i, _):
      off = base + i * _CH
      pltpu.sync_copy(inds_hbm.at[pl.ds(off, _CH)], idx_v)
      ck = pltpu.async_copy(kvtab_hbm.at[idx_v], kvbuf, semk)
      for j in range(_CH // 16):
        sl = pl.ds(j * 16, 16)
        idx16 = idx_v[sl]
        rowi = lax.shift_right_logical(
            jnp.full((16,), off + j * 16, jnp.int32) + iota16, 5)
        lid = iota16 * 4 + (j * 64)
        plsc.store_scatter(
            pbuf, [lid],
            plsc.load_gather(xtab, [idx16])
            - plsc.load_gather(qxtab, [rowi]))
        plsc.store_scatter(
            pbuf, [lid + 1],
            plsc.load_gather(ytab, [idx16])
            - plsc.load_gather(qytab, [rowi]))
        plsc.store_scatter(
            pbuf, [lid + 2],
            plsc.load_gather(ztab, [idx16])
            - plsc.load_gather(qztab, [rowi]))
        plsc.store_scatter(pbuf, [lid + 3], zero16)
      pltpu.sync_copy(pbuf, nbr_out.at[pl.ds(off * 4, _CH * 4)])
      ck.wait()
      pltpu.sync_copy(kvbuf, nkv_out.at[pl.ds(off, _CH)])
      return 0

    lax.fori_loop(0, _NCH, body, 0)

    @pl.when(wid < _NW - 7)
    def _():
      qbase = wid * _QROWS

      def qbody(i, _):
        off = qbase + i * _CH
        pltpu.sync_copy(inds0_hbm.at[pl.ds(off, _CH)], idx_v)
        pltpu.async_copy(qtab_hbm.at[idx_v], qbuf, semk).wait()
        pltpu.sync_copy(qbuf, qsel_out.at[pl.ds(off, _CH)])
        return 0

      lax.fori_loop(0, _QCH, qbody, 0)

  return gather_kernel(inds_flat, inds0, kvtab, qtab, px, py, pz,
                       qx, qy, qz)


def _qkv_body(s_ref, w_ref, b_ref, q_ref, kv_ref):
  acc = (
      jnp.dot(s_ref[...], w_ref[...], preferred_element_type=jnp.float32)
      + b_ref[...]
  )
  q_ref[...] = acc[:, :C]
  kb = lax.bitcast_convert_type(
      acc[:, C:2 * C].astype(jnp.bfloat16).astype(jnp.float32), jnp.int32)
  vb = lax.bitcast_convert_type(
      acc[:, 2 * C:].astype(jnp.bfloat16).astype(jnp.float32), jnp.int32)
  kv_ref[...] = lax.bitcast_convert_type(
      jnp.bitwise_or(lax.shift_right_logical(kb, 16), vb), jnp.float32)


def _qkv(s_feats, w_all, b_all):
  bm = 2000
  spec = pl.BlockSpec((bm, C), lambda i: (i, 0))
  return pl.pallas_call(
      _qkv_body,
      grid=(N // bm,),
      in_specs=[
          pl.BlockSpec((bm, C), lambda i: (i, 0)),
          pl.BlockSpec((C, 3 * C), lambda i: (0, 0)),
          pl.BlockSpec((1, 3 * C), lambda i: (0, 0)),
      ],
      out_specs=[spec, spec],
      out_shape=[
          jax.ShapeDtypeStruct((N, C), jnp.float32),
          jax.ShapeDtypeStruct((N, C), jnp.float32),
      ],
  )(s_feats, w_all, b_all)


def _leaky(x):
  return jnp.where(x >= 0, x, 0.1 * x)


def _unpack_k(kv):
  bits = lax.bitcast_convert_type(kv, jnp.int32)
  return lax.bitcast_convert_type(lax.shift_left(bits, 16), jnp.float32)


def _unpack_v(kv):
  bits = lax.bitcast_convert_type(kv, jnp.int32)
  return lax.bitcast_convert_type(
      jnp.bitwise_and(bits, jnp.int32(-65536)), jnp.float32)


def _edge_expand(wide, mask_ref, gmat_ref):
  rows = wide.shape[0]
  grp = BE // rows
  brd = jnp.broadcast_to(wide[:, None, :], (rows, grp, C))
  masked = (brd * mask_ref[...][None, :, :]).reshape(BE, C)
  return jnp.dot(masked, gmat_ref[...], preferred_element_type=jnp.float32)


def _geom(nbr_ref, sc1_ref, sh1_ref, m32_ref, g1_ref, wd2_ref, bd2_ref):
  y1 = _edge_expand(nbr_ref[...], m32_ref, g1_ref)
  hg = _leaky(y1 * sc1_ref[...] + sh1_ref[...])
  return (
      jnp.dot(hg, wd2_ref[...], preferred_element_type=jnp.float32)
      + bd2_ref[...]
  )


def _qk_edges(nk_ref, qsel_ref, geom):
  qsel_e = jnp.broadcast_to(qsel_ref[...][:, None, :],
                            (BM, H, C)).reshape(BE, C)
  return qsel_e - _unpack_k(nk_ref[...]) - geom


def _passA_body(nbr_ref, m32_ref, g1_ref, bd1_ref, sum_ref):
  y1 = _edge_expand(nbr_ref[...], m32_ref, g1_ref) + bd1_ref[...]
  s1 = jnp.sum(y1, axis=0)
  s2 = jnp.sum(y1 * y1, axis=0)

  @pl.when(pl.program_id(0) == 0)
  def _():
    sum_ref[...] = jnp.zeros_like(sum_ref)

  sum_ref[0, :] += s1
  sum_ref[1, :] += s2


def _passA(nbr4, m32, g1, bd1):
  return pl.pallas_call(
      _passA_body,
      grid=(GRID,),
      in_specs=[
          pl.BlockSpec((BE * 4 // C, C), lambda i: (i, 0)),
          pl.BlockSpec((H, C), lambda i: (0, 0)),
          pl.BlockSpec((C, C // 4), lambda i: (0, 0)),
          pl.BlockSpec((1, C // 4), lambda i: (0, 0)),
      ],
      out_specs=pl.BlockSpec((2, C // 4), lambda i: (0, 0)),
      out_shape=jax.ShapeDtypeStruct((2, C // 4), jnp.float32),
  )(nbr4, m32, g1, bd1)


_SMALL = lambda shape: pl.BlockSpec(shape, lambda i: (0, 0))


def _passB_body(nk_ref, nbr_ref, qsel_ref, m32_ref, g1_ref, sc1_ref,
                sh1_ref, wd2_ref, bd2_ref, sum_ref):
  geom = _geom(nbr_ref, sc1_ref, sh1_ref, m32_ref, g1_ref, wd2_ref, bd2_ref)
  qk = _qk_edges(nk_ref, qsel_ref, geom)
  s1 = jnp.sum(qk, axis=0)
  s2 = jnp.sum(qk * qk, axis=0)

  @pl.when(pl.program_id(0) == 0)
  def _():
    sum_ref[...] = jnp.zeros_like(sum_ref)

  sum_ref[0, :] += s1
  sum_ref[1, :] += s2


def _passB(nk, nbr4, q_sel, m32, g1, sc1, sh1, wd2, bd2):
  return pl.pallas_call(
      _passB_body,
      grid=(GRID,),
      in_specs=[
          pl.BlockSpec((BE, C), lambda i: (i, 0)),
          pl.BlockSpec((BE * 4 // C, C), lambda i: (i, 0)),
          pl.BlockSpec((BM, C), lambda i: (i, 0)),
          _SMALL((H, C)),
          _SMALL((C, C // 4)),
          _SMALL((1, C // 4)),
          _SMALL((1, C // 4)),
          _SMALL((C // 4, C)),
          _SMALL((1, C)),
      ],
      out_specs=pl.BlockSpec((2, C), lambda i: (0, 0)),
      out_shape=jax.ShapeDtypeStruct((2, C), jnp.float32),
  )(nk, nbr4, q_sel, m32, g1, sc1, sh1, wd2, bd2)


def _passC_body(nk_ref, nbr_ref, qsel_ref, m32_ref, g1_ref, sc1_ref,
                sh1_ref, wd2_ref, bd2_ref, sc2_ref, sh2_ref, wa1_ref,
                ba1_ref, tile_ref, m8_ref, y3_ref, sum_ref):
  geom = _geom(nbr_ref, sc1_ref, sh1_ref, m32_ref, g1_ref, wd2_ref, bd2_ref)
  qk = _qk_edges(nk_ref, qsel_ref, geom)
  a = _leaky(qk * sc2_ref[...] + sh2_ref[...])
  y3 = (
      jnp.dot(a, wa1_ref[...], preferred_element_type=jnp.float32)
      + ba1_ref[...]
  )
  y3sp = (
      jnp.dot(y3, tile_ref[...], preferred_element_type=jnp.float32)
      .reshape(BM, H, C) * m8_ref[...][None, :, :]
  ).reshape(BE * CPG // C, C // CPG, C)
  y3_ref[...] = jnp.sum(y3sp, axis=1)
  s1 = jnp.sum(y3, axis=0)
  s2 = jnp.sum(y3 * y3, axis=0)

  @pl.when(pl.program_id(0) == 0)
  def _():
    sum_ref[...] = jnp.zeros_like(sum_ref)

  sum_ref[0, :] += s1
  sum_ref[1, :] += s2


def _passC(nk, nbr4, q_sel, m32, g1, sc1, sh1, wd2, bd2, sc2, sh2, wa1,
           ba1, tile_mat, m8):
  return pl.pallas_call(
      _passC_body,
      grid=(GRID,),
      in_specs=[
          pl.BlockSpec((BE, C), lambda i: (i, 0)),
          pl.BlockSpec((BE * 4 // C, C), lambda i: (i, 0)),
          pl.BlockSpec((BM, C), lambda i: (i, 0)),
          _SMALL((H, C)),
          _SMALL((C, C // 4)),
          _SMALL((1, C // 4)),
          _SMALL((1, C // 4)),
          _SMALL((C // 4, C)),
          _SMALL((1, C)),
          _SMALL((1, C)),
          _SMALL((1, C)),
          _SMALL((C, CPG)),
          _SMALL((1, CPG)),
          _SMALL((CPG, C)),
          _SMALL((H, C)),
      ],
      out_specs=[
          pl.BlockSpec((BE * CPG // C, C), lambda i: (i, 0)),
          pl.BlockSpec((2, CPG), lambda i: (0, 0)),
      ],
      out_shape=[
          jax.ShapeDtypeStruct((EDGES * CPG // C, C), jnp.float32),
          jax.ShapeDtypeStruct((2, CPG), jnp.float32),
      ],
  )(nk, nbr4, q_sel, m32, g1, sc1, sh1, wd2, bd2, sc2, sh2, wa1,
    ba1, tile_mat, m8)


def _lane_butterfly(x, op):
  for sh in (CPG, 2 * CPG, 4 * CPG):
    x = op(x, pltpu.roll(x, sh, 1))
  return x


def _passD_body(nv_ref, nbr_ref, y3_ref, m32_ref, g1_ref, sc1_ref, sh1_ref,
                wd2_ref, bd2_ref, sc3_ref, sh3_ref, w2bd_ref, ba2_ref,
                m8_ref, gbig_ref, out_ref):
  geom = _geom(nbr_ref, sc1_ref, sh1_ref, m32_ref, g1_ref, wd2_ref, bd2_ref)
  vmg = _unpack_v(nv_ref[...]) - geom
  y3w = y3_ref[...]
  a2w = _leaky(y3w * sc3_ref[...] + sh3_ref[...])
  a3w = (
      jnp.dot(a2w, w2bd_ref[...], preferred_element_type=jnp.float32)
      + ba2_ref[...]
  )
  a34 = a3w.reshape(BM, H // 8, C)
  mx = _lane_butterfly(jnp.max(a34, axis=1), jnp.maximum)
  eb = jnp.exp(a3w - jnp.broadcast_to(
      mx[:, None, :], (BM, H // 8, C)).reshape(BM * H // 8, C))
  s = _lane_butterfly(jnp.sum(eb.reshape(BM, H // 8, C), axis=1),
                      jnp.add)
  rows = BM * H // 8
  ebrd = jnp.broadcast_to(eb[:, None, :], (rows, 8, C))
  emsk = (ebrd.reshape(BM, H, C) * m8_ref[...][None, :, :]).reshape(BE, C)
  et = jnp.dot(emsk, gbig_ref[...], preferred_element_type=jnp.float32)
  raw = jnp.sum((vmg * et).reshape(BM, H, C), axis=1)
  out_ref[...] = raw * (1.0 / s)


def _passD(nv, nbr4, y3, m32, g1, sc1, sh1, wd2, bd2, sc3t, sh3t, w2bd,
           ba2t, m8, gbig):
  return pl.pallas_call(
      _passD_body,
      grid=(GRID,),
      in_specs=[
          pl.BlockSpec((BE, C), lambda i: (i, 0)),
          pl.BlockSpec((BE * 4 // C, C), lambda i: (i, 0)),
          pl.BlockSpec((BE * CPG // C, C), lambda i: (i, 0)),
          _SMALL((H, C)),
          _SMALL((C, C // 4)),
          _SMALL((1, C // 4)),
          _SMALL((1, C // 4)),
          _SMALL((C // 4, C)),
          _SMALL((1, C)),
          _SMALL((1, C)),
          _SMALL((1, C)),
          _SMALL((C, C)),
          _SMALL((1, C)),
          _SMALL((H, C)),
          _SMALL((C, C)),
      ],
      out_specs=pl.BlockSpec((BM, C), lambda i: (i, 0)),
      out_shape=jax.ShapeDtypeStruct((N, C), jnp.float32),
  )(nv, nbr4, y3, m32, g1, sc1, sh1, wd2, bd2, sc3t, sh3t, w2bd, ba2t,
    m8, gbig)


def _bn_affine(sums, gamma, beta, bias):
  mean = sums[0] / EDGES
  var = sums[1] / EDGES - mean * mean
  rstd = lax.rsqrt(var + EPS)
  sc = rstd * gamma
  sh = (bias - mean) * sc + beta
  return sc.reshape(1, -1), sh.reshape(1, -1)


def kernel(q_pts, s_pts, s_feats, neighb_inds, Wq, bq, Wk, bk, Wv, bv, Wd1,
           bd1, g_d1, be_d1, Wd2, bd2, g_a0, be_a0, Wa1, ba1, g_a1, be_a1,
           Wa2, ba2):
  inds_flat = neighb_inds.reshape(-1)
  inds0 = neighb_inds[:, 0]
  px, py, pz = s_pts[:, 0], s_pts[:, 1], s_pts[:, 2]
  qx, qy, qz = q_pts[:, 0], q_pts[:, 1], q_pts[:, 2]
  wd1p = jnp.pad(Wd1, ((0, 1), (0, 0)))
  w_all = jnp.concatenate([Wq, Wk, Wv], axis=1)
  b_all = jnp.concatenate([bq, bk, bv]).reshape(1, 3 * C)
  tile_mat = jnp.tile(jnp.eye(CPG, dtype=jnp.float32), (1, C // CPG))
  g1 = jnp.tile(wd1p, (H, 1))
  lane = jnp.arange(C)
  m32 = (lane[None, :] // 4 == jnp.arange(H)[:, None]).astype(jnp.float32)
  m8 = (lane[None, :] // CPG
        == (jnp.arange(H) % 8)[:, None]).astype(jnp.float32)
  gbig = jnp.tile(jnp.eye(CPG, dtype=jnp.float32), (C // CPG, C // CPG))
  w2bd = jnp.kron(jnp.eye(C // CPG, dtype=jnp.float32), Wa2)

  qtab, kvtab = _qkv(s_feats, w_all, b_all)

  nkv, nbr_flat, q_sel = _sc_gather(inds_flat, inds0, kvtab, qtab,
                                    px, py, pz, qx, qy, qz)
  nbr4 = nbr_flat.reshape(EDGES * 4 // C, C)

  sumsA = _passA(nbr4, m32, g1, bd1.reshape(1, -1))
  sc1, sh1 = _bn_affine(sumsA, g_d1, be_d1, bd1)
  bd2r = bd2.reshape(1, C)

  sumsB = _passB(nkv, nbr4, q_sel, m32, g1, sc1, sh1, Wd2, bd2r)
  mean2 = sumsB[0] / EDGES
  var2 = sumsB[1] / EDGES - mean2 * mean2
  rstd2 = lax.rsqrt(var2 + EPS)
  sc2 = (rstd2 * g_a0).reshape(1, C)
  sh2 = (be_a0 - mean2 * rstd2 * g_a0).reshape(1, C)

  y3, sumsC = _passC(nkv, nbr4, q_sel, m32, g1, sc1, sh1, Wd2, bd2r, sc2,
                     sh2, Wa1, ba1.reshape(1, CPG), tile_mat, m8)
  sc3, sh3 = _bn_affine(sumsC, g_a1, be_a1, jnp.zeros((CPG,), jnp.float32))
  sc3t = jnp.tile(sc3, (1, C // CPG))
  sh3t = jnp.tile(sh3, (1, C // CPG))
  ba2t = jnp.tile(ba2, C // CPG).reshape(1, C)

  out = _passD(nkv, nbr4, y3, m32, g1, sc1, sh1, Wd2, bd2r, sc3t, sh3t,
               w2bd, ba2t, m8, gbig)
  return out


# speedup vs baseline: 4.3662x; 1.2021x over previous
import functools

import jax
import jax.numpy as jnp
from jax import lax
from jax.experimental import pallas as pl
from jax.experimental.pallas import tpu as pltpu
from jax.experimental.pallas import tpu_sc as plsc

N = 10000
H = 32
C = 128
CPG = 16
EDGES = N * H
BM = 200
BE = BM * H
GRID = N // BM
EPS = 1e-5


_NW = 32
_EPW = EDGES // _NW
_CH = 80
_NCH = _EPW // _CH
_QROWS = 400
_QCH = _QROWS // _CH


def _sc_gather(inds_flat, inds0, kvtab, qtab, px, py, pz, qx, qy, qz):
  mesh = plsc.VectorSubcoreMesh(core_axis_name="c", subcore_axis_name="s")

  @functools.partial(
      pl.kernel,
      out_type=[
          jax.ShapeDtypeStruct((EDGES, C), jnp.float32),
          jax.ShapeDtypeStruct((EDGES * 4,), jnp.float32),
          jax.ShapeDtypeStruct((N, C), jnp.float32),
      ],
      mesh=mesh,
      compiler_params=pltpu.CompilerParams(needs_layout_passes=False),
      scratch_types=[
          pltpu.VMEM((_EPW,), jnp.int32),
          pltpu.VMEM((_CH,), jnp.int32),
          pltpu.VMEM((_CH, C), jnp.float32),
          pltpu.VMEM((_CH, C), jnp.float32),
          pltpu.VMEM((_CH * 4,), jnp.float32),
          pltpu.VMEM((_CH * 4,), jnp.float32),
          pltpu.VMEM((N,), jnp.float32),
          pltpu.VMEM((N,), jnp.float32),
          pltpu.VMEM((N,), jnp.float32),
          pltpu.VMEM((N,), jnp.float32),
          pltpu.VMEM((N,), jnp.float32),
          pltpu.VMEM((N,), jnp.float32),
          pltpu.SemaphoreType.DMA,
          pltpu.SemaphoreType.DMA,
      ],
  )
  def gather_kernel(inds_hbm, inds0_hbm, kvtab_hbm, qtab_hbm,
                    px_hbm, py_hbm, pz_hbm, qx_hbm, qy_hbm, qz_hbm,
                    nkv_out, nbr_out, qsel_out,
                    idxall, idxq, bufa, bufb, pbufa, pbufb,
                    xtab, ytab, ztab, qxtab, qytab, qztab, sema, semb):
    wid = lax.axis_index("s") * 2 + lax.axis_index("c")
    base = wid * _EPW
    iota16 = lax.iota(jnp.int32, 16)
    zero16 = jnp.zeros((16,), jnp.float32)

    pltpu.sync_copy(inds_hbm.at[pl.ds(base, _EPW)], idxall)
    pltpu.sync_copy(px_hbm, xtab)
    pltpu.sync_copy(py_hbm, ytab)
    pltpu.sync_copy(pz_hbm, ztab)
    pltpu.sync_copy(qx_hbm, qxtab)
    pltpu.sync_copy(qy_hbm, qytab)
    pltpu.sync_copy(qz_hbm, qztab)

    def start(c, buf, sem):
      return pltpu.async_copy(
          kvtab_hbm.at[idxall.at[pl.ds(c * _CH, _CH)]], buf, sem)

    def wait_drain(buf, sem):
      pltpu.make_async_copy(
          kvtab_hbm.at[idxall.at[pl.ds(0, _CH)]], buf, sem).wait()

    def coords(c, pbuf):
      off = base + c * _CH
      for j in range(_CH // 16):
        idx16 = idxall[pl.ds(c * _CH + j * 16, 16)]
        rowi = lax.shift_right_logical(
            jnp.full((16,), off + j * 16, jnp.int32) + iota16, 5)
        lid = iota16 * 4 + (j * 64)
        plsc.store_scatter(
            pbuf, [lid],
            plsc.load_gather(xtab, [idx16])
            - plsc.load_gather(qxtab, [rowi]))
        plsc.store_scatter(
            pbuf, [lid + 1],
            plsc.load_gather(ytab, [idx16])
            - plsc.load_gather(qytab, [rowi]))
        plsc.store_scatter(
            pbuf, [lid + 2],
            plsc.load_gather(ztab, [idx16])
            - plsc.load_gather(qztab, [rowi]))
        plsc.store_scatter(pbuf, [lid + 3], zero16)
      pltpu.sync_copy(pbuf, nbr_out.at[pl.ds(off * 4, _CH * 4)])

    def finish(c, buf, sem):
      wait_drain(buf, sem)
      pltpu.sync_copy(buf, nkv_out.at[pl.ds(base + c * _CH, _CH)])

    start(0, bufa, sema)

    def pair(ii, _):
      c = 2 * ii
      start(c + 1, bufb, semb)
      coords(c, pbufa)
      finish(c, bufa, sema)
      start(c + 2, bufa, sema)
      coords(c + 1, pbufb)
      finish(c + 1, bufb, semb)
      return 0

    lax.fori_loop(0, (_NCH - 1) // 2, pair, 0)
    coords(_NCH - 1, pbufa)
    finish(_NCH - 1, bufa, sema)

    @pl.when(wid < _NW - 7)
    def _():
      qbase = wid * _QROWS

      def qbody(i, _):
        off = qbase + i * _CH
        pltpu.sync_copy(inds0_hbm.at[pl.ds(off, _CH)], idxq)
        pltpu.async_copy(qtab_hbm.at[idxq], bufa, sema).wait()
        pltpu.sync_copy(bufa, qsel_out.at[pl.ds(off, _CH)])
        return 0

      lax.fori_loop(0, _QCH, qbody, 0)

  return gather_kernel(inds_flat, inds0, kvtab, qtab, px, py, pz,
                       qx, qy, qz)


def _qkv_body(s_ref, w_ref, b_ref, q_ref, kv_ref):
  acc = (
      jnp.dot(s_ref[...], w_ref[...], preferred_element_type=jnp.float32)
      + b_ref[...]
  )
  q_ref[...] = acc[:, :C]
  kb = lax.bitcast_convert_type(
      acc[:, C:2 * C].astype(jnp.bfloat16).astype(jnp.float32), jnp.int32)
  vb = lax.bitcast_convert_type(
      acc[:, 2 * C:].astype(jnp.bfloat16).astype(jnp.float32), jnp.int32)
  kv_ref[...] = lax.bitcast_convert_type(
      jnp.bitwise_or(lax.shift_right_logical(kb, 16), vb), jnp.float32)


def _qkv(s_feats, w_all, b_all):
  bm = 2000
  spec = pl.BlockSpec((bm, C), lambda i: (i, 0))
  return pl.pallas_call(
      _qkv_body,
      grid=(N // bm,),
      in_specs=[
          pl.BlockSpec((bm, C), lambda i: (i, 0)),
          pl.BlockSpec((C, 3 * C), lambda i: (0, 0)),
          pl.BlockSpec((1, 3 * C), lambda i: (0, 0)),
      ],
      out_specs=[spec, spec],
      out_shape=[
          jax.ShapeDtypeStruct((N, C), jnp.float32),
          jax.ShapeDtypeStruct((N, C), jnp.float32),
      ],
  )(s_feats, w_all, b_all)


def _leaky(x):
  return jnp.where(x >= 0, x, 0.1 * x)


def _unpack_k(kv):
  bits = lax.bitcast_convert_type(kv, jnp.int32)
  return lax.bitcast_convert_type(lax.shift_left(bits, 16), jnp.float32)


def _unpack_v(kv):
  bits = lax.bitcast_convert_type(kv, jnp.int32)
  return lax.bitcast_convert_type(
      jnp.bitwise_and(bits, jnp.int32(-65536)), jnp.float32)


def _edge_expand(wide, mask_ref, gmat_ref):
  rows = wide.shape[0]
  grp = BE // rows
  brd = jnp.broadcast_to(wide[:, None, :], (rows, grp, C))
  masked = (brd * mask_ref[...][None, :, :]).reshape(BE, C)
  return jnp.dot(masked, gmat_ref[...], preferred_element_type=jnp.float32)


def _geom(nbr_ref, sc1_ref, sh1_ref, m32_ref, g1_ref, wd2_ref, bd2_ref):
  y1 = _edge_expand(nbr_ref[...], m32_ref, g1_ref)
  hg = _leaky(y1 * sc1_ref[...] + sh1_ref[...])
  return (
      jnp.dot(hg, wd2_ref[...], preferred_element_type=jnp.float32)
      + bd2_ref[...]
  )


def _qk_edges(nk_ref, qsel_ref, geom):
  qsel_e = jnp.broadcast_to(qsel_ref[...][:, None, :],
                            (BM, H, C)).reshape(BE, C)
  return qsel_e - _unpack_k(nk_ref[...]) - geom


def _passA_body(nbr_ref, m32_ref, g1_ref, bd1_ref, sum_ref):
  y1 = _edge_expand(nbr_ref[...], m32_ref, g1_ref) + bd1_ref[...]
  s1 = jnp.sum(y1, axis=0)
  s2 = jnp.sum(y1 * y1, axis=0)

  @pl.when(pl.program_id(0) == 0)
  def _():
    sum_ref[...] = jnp.zeros_like(sum_ref)

  sum_ref[0, :] += s1
  sum_ref[1, :] += s2


def _passA(nbr4, m32, g1, bd1):
  return pl.pallas_call(
      _passA_body,
      grid=(GRID,),
      in_specs=[
          pl.BlockSpec((BE * 4 // C, C), lambda i: (i, 0)),
          pl.BlockSpec((H, C), lambda i: (0, 0)),
          pl.BlockSpec((C, C // 4), lambda i: (0, 0)),
          pl.BlockSpec((1, C // 4), lambda i: (0, 0)),
      ],
      out_specs=pl.BlockSpec((2, C // 4), lambda i: (0, 0)),
      out_shape=jax.ShapeDtypeStruct((2, C // 4), jnp.float32),
  )(nbr4, m32, g1, bd1)


_SMALL = lambda shape: pl.BlockSpec(shape, lambda i: (0, 0))


def _passB_body(nk_ref, nbr_ref, qsel_ref, m32_ref, g1_ref, sc1_ref,
                sh1_ref, wd2_ref, bd2_ref, sum_ref):
  geom = _geom(nbr_ref, sc1_ref, sh1_ref, m32_ref, g1_ref, wd2_ref, bd2_ref)
  qk = _qk_edges(nk_ref, qsel_ref, geom)
  s1 = jnp.sum(qk, axis=0)
  s2 = jnp.sum(qk * qk, axis=0)

  @pl.when(pl.program_id(0) == 0)
  def _():
    sum_ref[...] = jnp.zeros_like(sum_ref)

  sum_ref[0, :] += s1
  sum_ref[1, :] += s2


def _passB(nk, nbr4, q_sel, m32, g1, sc1, sh1, wd2, bd2):
  return pl.pallas_call(
      _passB_body,
      grid=(GRID,),
      in_specs=[
          pl.BlockSpec((BE, C), lambda i: (i, 0)),
          pl.BlockSpec((BE * 4 // C, C), lambda i: (i, 0)),
          pl.BlockSpec((BM, C), lambda i: (i, 0)),
          _SMALL((H, C)),
          _SMALL((C, C // 4)),
          _SMALL((1, C // 4)),
          _SMALL((1, C // 4)),
          _SMALL((C // 4, C)),
          _SMALL((1, C)),
      ],
      out_specs=pl.BlockSpec((2, C), lambda i: (0, 0)),
      out_shape=jax.ShapeDtypeStruct((2, C), jnp.float32),
  )(nk, nbr4, q_sel, m32, g1, sc1, sh1, wd2, bd2)


def _passC_body(nk_ref, nbr_ref, qsel_ref, m32_ref, g1_ref, sc1_ref,
                sh1_ref, wd2_ref, bd2_ref, sc2_ref, sh2_ref, wa1_ref,
                ba1_ref, tile_ref, m8_ref, y3_ref, sum_ref):
  geom = _geom(nbr_ref, sc1_ref, sh1_ref, m32_ref, g1_ref, wd2_ref, bd2_ref)
  qk = _qk_edges(nk_ref, qsel_ref, geom)
  a = _leaky(qk * sc2_ref[...] + sh2_ref[...])
  y3 = (
      jnp.dot(a, wa1_ref[...], preferred_element_type=jnp.float32)
      + ba1_ref[...]
  )
  y3sp = (
      jnp.dot(y3, tile_ref[...], preferred_element_type=jnp.float32)
      .reshape(BM, H, C) * m8_ref[...][None, :, :]
  ).reshape(BE * CPG // C, C // CPG, C)
  y3_ref[...] = jnp.sum(y3sp, axis=1)
  s1 = jnp.sum(y3, axis=0)
  s2 = jnp.sum(y3 * y3, axis=0)

  @pl.when(pl.program_id(0) == 0)
  def _():
    sum_ref[...] = jnp.zeros_like(sum_ref)

  sum_ref[0, :] += s1
  sum_ref[1, :] += s2


def _passC(nk, nbr4, q_sel, m32, g1, sc1, sh1, wd2, bd2, sc2, sh2, wa1,
           ba1, tile_mat, m8):
  return pl.pallas_call(
      _passC_body,
      grid=(GRID,),
      in_specs=[
          pl.BlockSpec((BE, C), lambda i: (i, 0)),
          pl.BlockSpec((BE * 4 // C, C), lambda i: (i, 0)),
          pl.BlockSpec((BM, C), lambda i: (i, 0)),
          _SMALL((H, C)),
          _SMALL((C, C // 4)),
          _SMALL((1, C // 4)),
          _SMALL((1, C // 4)),
          _SMALL((C // 4, C)),
          _SMALL((1, C)),
          _SMALL((1, C)),
          _SMALL((1, C)),
          _SMALL((C, CPG)),
          _SMALL((1, CPG)),
          _SMALL((CPG, C)),
          _SMALL((H, C)),
      ],
      out_specs=[
          pl.BlockSpec((BE * CPG // C, C), lambda i: (i, 0)),
          pl.BlockSpec((2, CPG), lambda i: (0, 0)),
      ],
      out_shape=[
          jax.ShapeDtypeStruct((EDGES * CPG // C, C), jnp.float32),
          jax.ShapeDtypeStruct((2, CPG), jnp.float32),
      ],
  )(nk, nbr4, q_sel, m32, g1, sc1, sh1, wd2, bd2, sc2, sh2, wa1,
    ba1, tile_mat, m8)


def _lane_butterfly(x, op):
  for sh in (CPG, 2 * CPG, 4 * CPG):
    x = op(x, pltpu.roll(x, sh, 1))
  return x


def _passD_body(nv_ref, nbr_ref, y3_ref, m32_ref, g1_ref, sc1_ref, sh1_ref,
                wd2_ref, bd2_ref, sc3_ref, sh3_ref, w2bd_ref, ba2_ref,
                m8_ref, gbig_ref, out_ref):
  geom = _geom(nbr_ref, sc1_ref, sh1_ref, m32_ref, g1_ref, wd2_ref, bd2_ref)
  vmg = _unpack_v(nv_ref[...]) - geom
  y3w = y3_ref[...]
  a2w = _leaky(y3w * sc3_ref[...] + sh3_ref[...])
  a3w = (
      jnp.dot(a2w, w2bd_ref[...], preferred_element_type=jnp.float32)
      + ba2_ref[...]
  )
  a34 = a3w.reshape(BM, H // 8, C)
  mx = _lane_butterfly(jnp.max(a34, axis=1), jnp.maximum)
  eb = jnp.exp(a3w - jnp.broadcast_to(
      mx[:, None, :], (BM, H // 8, C)).reshape(BM * H // 8, C))
  s = _lane_butterfly(jnp.sum(eb.reshape(BM, H // 8, C), axis=1),
                      jnp.add)
  rows = BM * H // 8
  ebrd = jnp.broadcast_to(eb[:, None, :], (rows, 8, C))
  emsk = (ebrd.reshape(BM, H, C) * m8_ref[...][None, :, :]).reshape(BE, C)
  et = jnp.dot(emsk, gbig_ref[...], preferred_element_type=jnp.float32)
  raw = jnp.sum((vmg * et).reshape(BM, H, C), axis=1)
  out_ref[...] = raw * (1.0 / s)


def _passD(nv, nbr4, y3, m32, g1, sc1, sh1, wd2, bd2, sc3t, sh3t, w2bd,
           ba2t, m8, gbig):
  return pl.pallas_call(
      _passD_body,
      grid=(GRID,),
      in_specs=[
          pl.BlockSpec((BE, C), lambda i: (i, 0)),
          pl.BlockSpec((BE * 4 // C, C), lambda i: (i, 0)),
          pl.BlockSpec((BE * CPG // C, C), lambda i: (i, 0)),
          _SMALL((H, C)),
          _SMALL((C, C // 4)),
          _SMALL((1, C // 4)),
          _SMALL((1, C // 4)),
          _SMALL((C // 4, C)),
          _SMALL((1, C)),
          _SMALL((1, C)),
          _SMALL((1, C)),
          _SMALL((C, C)),
          _SMALL((1, C)),
          _SMALL((H, C)),
          _SMALL((C, C)),
      ],
      out_specs=pl.BlockSpec((BM, C), lambda i: (i, 0)),
      out_shape=jax.ShapeDtypeStruct((N, C), jnp.float32),
  )(nv, nbr4, y3, m32, g1, sc1, sh1, wd2, bd2, sc3t, sh3t, w2bd, ba2t,
    m8, gbig)


def _bn_affine(sums, gamma, beta, bias):
  mean = sums[0] / EDGES
  var = sums[1] / EDGES - mean * mean
  rstd = lax.rsqrt(var + EPS)
  sc = rstd * gamma
  sh = (bias - mean) * sc + beta
  return sc.reshape(1, -1), sh.reshape(1, -1)


def kernel(q_pts, s_pts, s_feats, neighb_inds, Wq, bq, Wk, bk, Wv, bv, Wd1,
           bd1, g_d1, be_d1, Wd2, bd2, g_a0, be_a0, Wa1, ba1, g_a1, be_a1,
           Wa2, ba2):
  inds_flat = neighb_inds.reshape(-1)
  inds0 = neighb_inds[:, 0]
  px, py, pz = s_pts[:, 0], s_pts[:, 1], s_pts[:, 2]
  qx, qy, qz = q_pts[:, 0], q_pts[:, 1], q_pts[:, 2]
  wd1p = jnp.pad(Wd1, ((0, 1), (0, 0)))
  w_all = jnp.concatenate([Wq, Wk, Wv], axis=1)
  b_all = jnp.concatenate([bq, bk, bv]).reshape(1, 3 * C)
  tile_mat = jnp.tile(jnp.eye(CPG, dtype=jnp.float32), (1, C // CPG))
  g1 = jnp.tile(wd1p, (H, 1))
  lane = jnp.arange(C)
  m32 = (lane[None, :] // 4 == jnp.arange(H)[:, None]).astype(jnp.float32)
  m8 = (lane[None, :] // CPG
        == (jnp.arange(H) % 8)[:, None]).astype(jnp.float32)
  gbig = jnp.tile(jnp.eye(CPG, dtype=jnp.float32), (C // CPG, C // CPG))
  w2bd = jnp.kron(jnp.eye(C // CPG, dtype=jnp.float32), Wa2)

  qtab, kvtab = _qkv(s_feats, w_all, b_all)

  nkv, nbr_flat, q_sel = _sc_gather(inds_flat, inds0, kvtab, qtab,
                                    px, py, pz, qx, qy, qz)
  nbr4 = nbr_flat.reshape(EDGES * 4 // C, C)

  sumsA = _passA(nbr4, m32, g1, bd1.reshape(1, -1))
  sc1, sh1 = _bn_affine(sumsA, g_d1, be_d1, bd1)
  bd2r = bd2.reshape(1, C)

  sumsB = _passB(nkv, nbr4, q_sel, m32, g1, sc1, sh1, Wd2, bd2r)
  mean2 = sumsB[0] / EDGES
  var2 = sumsB[1] / EDGES - mean2 * mean2
  rstd2 = lax.rsqrt(var2 + EPS)
  sc2 = (rstd2 * g_a0).reshape(1, C)
  sh2 = (be_a0 - mean2 * rstd2 * g_a0).reshape(1, C)

  y3, sumsC = _passC(nkv, nbr4, q_sel, m32, g1, sc1, sh1, Wd2, bd2r, sc2,
                     sh2, Wa1, ba1.reshape(1, CPG), tile_mat, m8)
  sc3, sh3 = _bn_affine(sumsC, g_a1, be_a1, jnp.zeros((CPG,), jnp.float32))
  sc3t = jnp.tile(sc3, (1, C // CPG))
  sh3t = jnp.tile(sh3, (1, C // CPG))
  ba2t = jnp.tile(ba2, C // CPG).reshape(1, C)

  out = _passD(nkv, nbr4, y3, m32, g1, sc1, sh1, Wd2, bd2r, sc3t, sh3t,
               w2bd, ba2t, m8, gbig)
  return out


# speedup vs baseline: 4.5567x; 1.0436x over previous
import functools

import jax
import jax.numpy as jnp
from jax import lax
from jax.experimental import pallas as pl
from jax.experimental.pallas import tpu as pltpu
from jax.experimental.pallas import tpu_sc as plsc

N = 10000
H = 32
C = 128
CPG = 16
EDGES = N * H
BM = 400
BE = BM * H
GRID = N // BM
EPS = 1e-5


_NW = 32
_EPW = EDGES // _NW
_CH = 80
_NCH = _EPW // _CH
_QROWS = 400
_QCH = _QROWS // _CH


def _sc_gather(inds_flat, inds0, kvtab, qtab, px, py, pz, qx, qy, qz):
  mesh = plsc.VectorSubcoreMesh(core_axis_name="c", subcore_axis_name="s")

  @functools.partial(
      pl.kernel,
      out_type=[
          jax.ShapeDtypeStruct((EDGES, C), jnp.float32),
          jax.ShapeDtypeStruct((EDGES * 4,), jnp.float32),
          jax.ShapeDtypeStruct((N, C), jnp.float32),
      ],
      mesh=mesh,
      compiler_params=pltpu.CompilerParams(needs_layout_passes=False),
      scratch_types=[
          pltpu.VMEM((_EPW,), jnp.int32),
          pltpu.VMEM((_CH,), jnp.int32),
          pltpu.VMEM((_CH, C), jnp.float32),
          pltpu.VMEM((_CH, C), jnp.float32),
          pltpu.VMEM((_CH * 4,), jnp.float32),
          pltpu.VMEM((_CH * 4,), jnp.float32),
          pltpu.VMEM((N,), jnp.float32),
          pltpu.VMEM((N,), jnp.float32),
          pltpu.VMEM((N,), jnp.float32),
          pltpu.VMEM((N,), jnp.float32),
          pltpu.VMEM((N,), jnp.float32),
          pltpu.VMEM((N,), jnp.float32),
          pltpu.SemaphoreType.DMA,
          pltpu.SemaphoreType.DMA,
      ],
  )
  def gather_kernel(inds_hbm, inds0_hbm, kvtab_hbm, qtab_hbm,
                    px_hbm, py_hbm, pz_hbm, qx_hbm, qy_hbm, qz_hbm,
                    nkv_out, nbr_out, qsel_out,
                    idxall, idxq, bufa, bufb, pbufa, pbufb,
                    xtab, ytab, ztab, qxtab, qytab, qztab, sema, semb):
    wid = lax.axis_index("s") * 2 + lax.axis_index("c")
    base = wid * _EPW
    iota16 = lax.iota(jnp.int32, 16)
    zero16 = jnp.zeros((16,), jnp.float32)

    pltpu.sync_copy(inds_hbm.at[pl.ds(base, _EPW)], idxall)
    pltpu.sync_copy(px_hbm, xtab)
    pltpu.sync_copy(py_hbm, ytab)
    pltpu.sync_copy(pz_hbm, ztab)
    pltpu.sync_copy(qx_hbm, qxtab)
    pltpu.sync_copy(qy_hbm, qytab)
    pltpu.sync_copy(qz_hbm, qztab)

    def start(c, buf, sem):
      return pltpu.async_copy(
          kvtab_hbm.at[idxall.at[pl.ds(c * _CH, _CH)]], buf, sem)

    def wait_drain(buf, sem):
      pltpu.make_async_copy(
          kvtab_hbm.at[idxall.at[pl.ds(0, _CH)]], buf, sem).wait()

    def coords(c, pbuf):
      off = base + c * _CH
      for j in range(_CH // 16):
        idx16 = idxall[pl.ds(c * _CH + j * 16, 16)]
        rowi = lax.shift_right_logical(
            jnp.full((16,), off + j * 16, jnp.int32) + iota16, 5)
        lid = iota16 * 4 + (j * 64)
        plsc.store_scatter(
            pbuf, [lid],
            plsc.load_gather(xtab, [idx16])
            - plsc.load_gather(qxtab, [rowi]))
        plsc.store_scatter(
            pbuf, [lid + 1],
            plsc.load_gather(ytab, [idx16])
            - plsc.load_gather(qytab, [rowi]))
        plsc.store_scatter(
            pbuf, [lid + 2],
            plsc.load_gather(ztab, [idx16])
            - plsc.load_gather(qztab, [rowi]))
        plsc.store_scatter(pbuf, [lid + 3], zero16)
      pltpu.sync_copy(pbuf, nbr_out.at[pl.ds(off * 4, _CH * 4)])

    def finish(c, buf, sem):
      wait_drain(buf, sem)
      pltpu.sync_copy(buf, nkv_out.at[pl.ds(base + c * _CH, _CH)])

    start(0, bufa, sema)

    def pair(ii, _):
      c = 2 * ii
      start(c + 1, bufb, semb)
      coords(c, pbufa)
      finish(c, bufa, sema)
      start(c + 2, bufa, sema)
      coords(c + 1, pbufb)
      finish(c + 1, bufb, semb)
      return 0

    lax.fori_loop(0, (_NCH - 1) // 2, pair, 0)
    coords(_NCH - 1, pbufa)
    finish(_NCH - 1, bufa, sema)

    def qbody(t, _):
      qc = wid + _NW * t

      @pl.when(qc < N // _CH)
      def _():
        off = qc * _CH
        pltpu.sync_copy(inds0_hbm.at[pl.ds(off, _CH)], idxq)
        pltpu.async_copy(qtab_hbm.at[idxq], bufa, sema).wait()
        pltpu.sync_copy(bufa, qsel_out.at[pl.ds(off, _CH)])

      return 0

    lax.fori_loop(0, 4, qbody, 0)

  return gather_kernel(inds_flat, inds0, kvtab, qtab, px, py, pz,
                       qx, qy, qz)


def _qkv_body(s_ref, w_ref, b_ref, q_ref, kv_ref):
  acc = (
      jnp.dot(s_ref[...], w_ref[...], preferred_element_type=jnp.float32)
      + b_ref[...]
  )
  q_ref[...] = acc[:, :C]
  kb = lax.bitcast_convert_type(
      acc[:, C:2 * C].astype(jnp.bfloat16).astype(jnp.float32), jnp.int32)
  vb = lax.bitcast_convert_type(
      acc[:, 2 * C:].astype(jnp.bfloat16).astype(jnp.float32), jnp.int32)
  kv_ref[...] = lax.bitcast_convert_type(
      jnp.bitwise_or(lax.shift_right_logical(kb, 16), vb), jnp.float32)


def _qkv(s_feats, w_all, b_all):
  bm = 2000
  spec = pl.BlockSpec((bm, C), lambda i: (i, 0))
  return pl.pallas_call(
      _qkv_body,
      grid=(N // bm,),
      in_specs=[
          pl.BlockSpec((bm, C), lambda i: (i, 0)),
          pl.BlockSpec((C, 3 * C), lambda i: (0, 0)),
          pl.BlockSpec((1, 3 * C), lambda i: (0, 0)),
      ],
      out_specs=[spec, spec],
      out_shape=[
          jax.ShapeDtypeStruct((N, C), jnp.float32),
          jax.ShapeDtypeStruct((N, C), jnp.float32),
      ],
  )(s_feats, w_all, b_all)


def _leaky(x):
  return jnp.where(x >= 0, x, 0.1 * x)


def _unpack_k(kv):
  bits = lax.bitcast_convert_type(kv, jnp.int32)
  return lax.bitcast_convert_type(lax.shift_left(bits, 16), jnp.float32)


def _unpack_v(kv):
  bits = lax.bitcast_convert_type(kv, jnp.int32)
  return lax.bitcast_convert_type(
      jnp.bitwise_and(bits, jnp.int32(-65536)), jnp.float32)


def _edge_expand(wide, mask_ref, gmat_ref):
  rows = wide.shape[0]
  grp = BE // rows
  brd = jnp.broadcast_to(wide[:, None, :], (rows, grp, C))
  masked = (brd * mask_ref[...][None, :, :]).reshape(BE, C)
  return jnp.dot(masked, gmat_ref[...], preferred_element_type=jnp.float32)


def _geom(nbr_ref, sc1_ref, sh1_ref, m32_ref, g1_ref, wd2_ref, bd2_ref):
  y1 = _edge_expand(nbr_ref[...], m32_ref, g1_ref)
  hg = _leaky(y1 * sc1_ref[...] + sh1_ref[...])
  return (
      jnp.dot(hg, wd2_ref[...], preferred_element_type=jnp.float32)
      + bd2_ref[...]
  )


def _qk_edges(nk_ref, qsel_ref, geom):
  qsel_e = jnp.broadcast_to(qsel_ref[...][:, None, :],
                            (BM, H, C)).reshape(BE, C)
  return qsel_e - _unpack_k(nk_ref[...]) - geom


def _passA_body(nbr_ref, m32_ref, g1_ref, bd1_ref, sum_ref):
  y1 = _edge_expand(nbr_ref[...], m32_ref, g1_ref) + bd1_ref[...]
  s1 = jnp.sum(y1, axis=0)
  s2 = jnp.sum(y1 * y1, axis=0)

  @pl.when(pl.program_id(0) == 0)
  def _():
    sum_ref[...] = jnp.zeros_like(sum_ref)

  sum_ref[0, :] += s1
  sum_ref[1, :] += s2


def _passA(nbr4, m32, g1, bd1):
  return pl.pallas_call(
      _passA_body,
      grid=(GRID,),
      in_specs=[
          pl.BlockSpec((BE * 4 // C, C), lambda i: (i, 0)),
          pl.BlockSpec((H, C), lambda i: (0, 0)),
          pl.BlockSpec((C, C // 4), lambda i: (0, 0)),
          pl.BlockSpec((1, C // 4), lambda i: (0, 0)),
      ],
      out_specs=pl.BlockSpec((2, C // 4), lambda i: (0, 0)),
      out_shape=jax.ShapeDtypeStruct((2, C // 4), jnp.float32),
  )(nbr4, m32, g1, bd1)


_SMALL = lambda shape: pl.BlockSpec(shape, lambda i: (0, 0))


def _passB_body(nk_ref, nbr_ref, qsel_ref, m32_ref, g1_ref, sc1_ref,
                sh1_ref, wd2_ref, bd2_ref, sum_ref):
  geom = _geom(nbr_ref, sc1_ref, sh1_ref, m32_ref, g1_ref, wd2_ref, bd2_ref)
  qk = _qk_edges(nk_ref, qsel_ref, geom)
  s1 = jnp.sum(qk, axis=0)
  s2 = jnp.sum(qk * qk, axis=0)

  @pl.when(pl.program_id(0) == 0)
  def _():
    sum_ref[...] = jnp.zeros_like(sum_ref)

  sum_ref[0, :] += s1
  sum_ref[1, :] += s2


def _passB(nk, nbr4, q_sel, m32, g1, sc1, sh1, wd2, bd2):
  return pl.pallas_call(
      _passB_body,
      grid=(GRID,),
      in_specs=[
          pl.BlockSpec((BE, C), lambda i: (i, 0)),
          pl.BlockSpec((BE * 4 // C, C), lambda i: (i, 0)),
          pl.BlockSpec((BM, C), lambda i: (i, 0)),
          _SMALL((H, C)),
          _SMALL((C, C // 4)),
          _SMALL((1, C // 4)),
          _SMALL((1, C // 4)),
          _SMALL((C // 4, C)),
          _SMALL((1, C)),
      ],
      out_specs=pl.BlockSpec((2, C), lambda i: (0, 0)),
      out_shape=jax.ShapeDtypeStruct((2, C), jnp.float32),
  )(nk, nbr4, q_sel, m32, g1, sc1, sh1, wd2, bd2)


def _passC_body(nk_ref, nbr_ref, qsel_ref, m32_ref, g1_ref, sc1_ref,
                sh1_ref, wd2_ref, bd2_ref, sc2_ref, sh2_ref, wa1_ref,
                ba1_ref, tile_ref, m8_ref, y3_ref, sum_ref):
  geom = _geom(nbr_ref, sc1_ref, sh1_ref, m32_ref, g1_ref, wd2_ref, bd2_ref)
  qk = _qk_edges(nk_ref, qsel_ref, geom)
  a = _leaky(qk * sc2_ref[...] + sh2_ref[...])
  y3 = (
      jnp.dot(a, wa1_ref[...], preferred_element_type=jnp.float32)
      + ba1_ref[...]
  )
  y3sp = (
      jnp.dot(y3, tile_ref[...], preferred_element_type=jnp.float32)
      .reshape(BM, H, C) * m8_ref[...][None, :, :]
  ).reshape(BE * CPG // C, C // CPG, C)
  y3_ref[...] = jnp.sum(y3sp, axis=1)
  s1 = jnp.sum(y3, axis=0)
  s2 = jnp.sum(y3 * y3, axis=0)

  @pl.when(pl.program_id(0) == 0)
  def _():
    sum_ref[...] = jnp.zeros_like(sum_ref)

  sum_ref[0, :] += s1
  sum_ref[1, :] += s2


def _passC(nk, nbr4, q_sel, m32, g1, sc1, sh1, wd2, bd2, sc2, sh2, wa1,
           ba1, tile_mat, m8):
  return pl.pallas_call(
      _passC_body,
      grid=(GRID,),
      in_specs=[
          pl.BlockSpec((BE, C), lambda i: (i, 0)),
          pl.BlockSpec((BE * 4 // C, C), lambda i: (i, 0)),
          pl.BlockSpec((BM, C), lambda i: (i, 0)),
          _SMALL((H, C)),
          _SMALL((C, C // 4)),
          _SMALL((1, C // 4)),
          _SMALL((1, C // 4)),
          _SMALL((C // 4, C)),
          _SMALL((1, C)),
          _SMALL((1, C)),
          _SMALL((1, C)),
          _SMALL((C, CPG)),
          _SMALL((1, CPG)),
          _SMALL((CPG, C)),
          _SMALL((H, C)),
      ],
      out_specs=[
          pl.BlockSpec((BE * CPG // C, C), lambda i: (i, 0)),
          pl.BlockSpec((2, CPG), lambda i: (0, 0)),
      ],
      out_shape=[
          jax.ShapeDtypeStruct((EDGES * CPG // C, C), jnp.float32),
          jax.ShapeDtypeStruct((2, CPG), jnp.float32),
      ],
  )(nk, nbr4, q_sel, m32, g1, sc1, sh1, wd2, bd2, sc2, sh2, wa1,
    ba1, tile_mat, m8)


def _lane_butterfly(x, op):
  for sh in (CPG, 2 * CPG, 4 * CPG):
    x = op(x, pltpu.roll(x, sh, 1))
  return x


def _passD_body(nv_ref, nbr_ref, y3_ref, m32_ref, g1_ref, sc1_ref, sh1_ref,
                wd2_ref, bd2_ref, sc3_ref, sh3_ref, w2bd_ref, ba2_ref,
                m8_ref, gbig_ref, out_ref):
  geom = _geom(nbr_ref, sc1_ref, sh1_ref, m32_ref, g1_ref, wd2_ref, bd2_ref)
  vmg = _unpack_v(nv_ref[...]) - geom
  y3w = y3_ref[...]
  a2w = _leaky(y3w * sc3_ref[...] + sh3_ref[...])
  a3w = (
      jnp.dot(a2w, w2bd_ref[...], preferred_element_type=jnp.float32)
      + ba2_ref[...]
  )
  a34 = a3w.reshape(BM, H // 8, C)
  mx = _lane_butterfly(jnp.max(a34, axis=1), jnp.maximum)
  eb = jnp.exp(a3w - jnp.broadcast_to(
      mx[:, None, :], (BM, H // 8, C)).reshape(BM * H // 8, C))
  s = _lane_butterfly(jnp.sum(eb.reshape(BM, H // 8, C), axis=1),
                      jnp.add)
  rows = BM * H // 8
  ebrd = jnp.broadcast_to(eb[:, None, :], (rows, 8, C))
  emsk = (ebrd.reshape(BM, H, C) * m8_ref[...][None, :, :]).reshape(BE, C)
  et = jnp.dot(emsk, gbig_ref[...], preferred_element_type=jnp.float32)
  raw = jnp.sum((vmg * et).reshape(BM, H, C), axis=1)
  out_ref[...] = raw * (1.0 / s)


def _passD(nv, nbr4, y3, m32, g1, sc1, sh1, wd2, bd2, sc3t, sh3t, w2bd,
           ba2t, m8, gbig):
  return pl.pallas_call(
      _passD_body,
      grid=(GRID,),
      in_specs=[
          pl.BlockSpec((BE, C), lambda i: (i, 0)),
          pl.BlockSpec((BE * 4 // C, C), lambda i: (i, 0)),
          pl.BlockSpec((BE * CPG // C, C), lambda i: (i, 0)),
          _SMALL((H, C)),
          _SMALL((C, C // 4)),
          _SMALL((1, C // 4)),
          _SMALL((1, C // 4)),
          _SMALL((C // 4, C)),
          _SMALL((1, C)),
          _SMALL((1, C)),
          _SMALL((1, C)),
          _SMALL((C, C)),
          _SMALL((1, C)),
          _SMALL((H, C)),
          _SMALL((C, C)),
      ],
      out_specs=pl.BlockSpec((BM, C), lambda i: (i, 0)),
      out_shape=jax.ShapeDtypeStruct((N, C), jnp.float32),
  )(nv, nbr4, y3, m32, g1, sc1, sh1, wd2, bd2, sc3t, sh3t, w2bd, ba2t,
    m8, gbig)


def _bn_affine(sums, gamma, beta, bias):
  mean = sums[0] / EDGES
  var = sums[1] / EDGES - mean * mean
  rstd = lax.rsqrt(var + EPS)
  sc = rstd * gamma
  sh = (bias - mean) * sc + beta
  return sc.reshape(1, -1), sh.reshape(1, -1)


def kernel(q_pts, s_pts, s_feats, neighb_inds, Wq, bq, Wk, bk, Wv, bv, Wd1,
           bd1, g_d1, be_d1, Wd2, bd2, g_a0, be_a0, Wa1, ba1, g_a1, be_a1,
           Wa2, ba2):
  inds_flat = neighb_inds.reshape(-1)
  inds0 = neighb_inds[:, 0]
  px, py, pz = s_pts[:, 0], s_pts[:, 1], s_pts[:, 2]
  qx, qy, qz = q_pts[:, 0], q_pts[:, 1], q_pts[:, 2]
  wd1p = jnp.pad(Wd1, ((0, 1), (0, 0)))
  w_all = jnp.concatenate([Wq, Wk, Wv], axis=1)
  b_all = jnp.concatenate([bq, bk, bv]).reshape(1, 3 * C)
  tile_mat = jnp.tile(jnp.eye(CPG, dtype=jnp.float32), (1, C // CPG))
  g1 = jnp.tile(wd1p, (H, 1))
  lane = jnp.arange(C)
  m32 = (lane[None, :] // 4 == jnp.arange(H)[:, None]).astype(jnp.float32)
  m8 = (lane[None, :] // CPG
        == (jnp.arange(H) % 8)[:, None]).astype(jnp.float32)
  gbig = jnp.tile(jnp.eye(CPG, dtype=jnp.float32), (C // CPG, C // CPG))
  w2bd = jnp.kron(jnp.eye(C // CPG, dtype=jnp.float32), Wa2)

  qtab, kvtab = _qkv(s_feats, w_all, b_all)

  nkv, nbr_flat, q_sel = _sc_gather(inds_flat, inds0, kvtab, qtab,
                                    px, py, pz, qx, qy, qz)
  nbr4 = nbr_flat.reshape(EDGES * 4 // C, C)

  sumsA = _passA(nbr4, m32, g1, bd1.reshape(1, -1))
  sc1, sh1 = _bn_affine(sumsA, g_d1, be_d1, bd1)
  bd2r = bd2.reshape(1, C)

  sumsB = _passB(nkv, nbr4, q_sel, m32, g1, sc1, sh1, Wd2, bd2r)
  mean2 = sumsB[0] / EDGES
  var2 = sumsB[1] / EDGES - mean2 * mean2
  rstd2 = lax.rsqrt(var2 + EPS)
  sc2 = (rstd2 * g_a0).reshape(1, C)
  sh2 = (be_a0 - mean2 * rstd2 * g_a0).reshape(1, C)

  y3, sumsC = _passC(nkv, nbr4, q_sel, m32, g1, sc1, sh1, Wd2, bd2r, sc2,
                     sh2, Wa1, ba1.reshape(1, CPG), tile_mat, m8)
  sc3, sh3 = _bn_affine(sumsC, g_a1, be_a1, jnp.zeros((CPG,), jnp.float32))
  sc3t = jnp.tile(sc3, (1, C // CPG))
  sh3t = jnp.tile(sh3, (1, C // CPG))
  ba2t = jnp.tile(ba2, C // CPG).reshape(1, C)

  out = _passD(nkv, nbr4, y3, m32, g1, sc1, sh1, Wd2, bd2r, sc3t, sh3t,
               w2bd, ba2t, m8, gbig)
  return out
